# Initial kernel scaffold; baseline (speedup 1.0000x reference)
import jax
import jax.numpy as jnp
from jax.experimental import pallas as pl


def kernel(node_features, edge_attr, enc_W, enc_b, enc_g, enc_beta, conv_W, conv_b, norm_g, norm_b, mlp_W1, mlp_b1, mlp_W2, mlp_b2, edge_index):
    raise NotImplementedError("write your pallas kernel here")


# speedup vs baseline: 5.1780x; 5.1780x over previous
import functools

import jax
import jax.numpy as jnp
from jax import lax
from jax.experimental import pallas as pl
from jax.experimental.pallas import tpu as pltpu
from jax.experimental.pallas import tpu_sc as plsc

N = 50000
E = 800000
H = 64
D_IN = 128
NC = 2
NS = 16
HALF = N // NC
DUMMY = HALF
SPROWS = HALF + 16
EKA = 80
EK = 128
CH = 200
NCHUNK = HALF // CH

_vmesh = plsc.VectorSubcoreMesh(core_axis_name="c", subcore_axis_name="s")
_sc_params = pltpu.CompilerParams(use_tc_tiling_on_sc=False)


def _sc_deg(dst_hbm, deg_hbm, zbuf, ones_v, idx_v, loc_v, acc):
    cid = lax.axis_index("c")
    sid = lax.axis_index("s")
    base = cid * HALF

    z16 = jnp.zeros((16,), jnp.float32)

    @pl.loop(0, CH)
    def _(r):
        zbuf[r, :] = z16

    @pl.loop(0, NCHUNK // NS + 1)
    def _(t):
        idx = sid + t * NS

        @pl.when(idx < NCHUNK)
        def _():
            pltpu.sync_copy(zbuf, acc.at[pl.ds(idx * CH, CH)])

    lane = lax.iota(jnp.int32, 16)
    unit = jnp.where(lane == 0, jnp.float32(1.0), jnp.float32(0.0))

    @pl.loop(0, EKA)
    def _(r):
        ones_v[r, :] = unit

    plsc.subcore_barrier()

    eps = E // NS

    @pl.loop(0, eps // EKA)
    def _(b):
        e0 = sid * eps + b * EKA
        pltpu.sync_copy(dst_hbm.at[pl.ds(e0, EKA)], idx_v)

        @pl.loop(0, EKA, step=16)
        def _(i):
            d = idx_v[pl.ds(i, 16)]
            loc = d - base
            inb = (loc >= 0) & (loc < HALF)
            loc_v[pl.ds(i, 16)] = jnp.where(inb, loc, DUMMY)

        pltpu.sync_copy(ones_v, acc.at[loc_v], add=True)

    plsc.subcore_barrier()

    @pl.loop(0, NCHUNK // NS + 1)
    def _(t):
        idx = sid + t * NS

        @pl.when(idx < NCHUNK)
        def _():
            pltpu.sync_copy(acc.at[pl.ds(idx * CH, CH)],
                            deg_hbm.at[pl.ds(base + idx * CH, CH)])


def _deg(dst):
    k = pl.kernel(
        _sc_deg,
        out_type=jax.ShapeDtypeStruct((N, 16), jnp.float32),
        mesh=_vmesh,
        compiler_params=_sc_params,
        scratch_types=[
            pltpu.VMEM((CH, 16), jnp.float32),
            pltpu.VMEM((EKA, 16), jnp.float32),
            pltpu.VMEM((EKA,), jnp.int32),
            pltpu.VMEM((EKA,), jnp.int32),
            pltpu.VMEM_SHARED((SPROWS, 16), jnp.float32),
        ],
    )
    return k(dst)


def _sc_agg(hp_hbm, src_hbm, dst_hbm, agg_hbm, sidx_v, didx_v, loc_v, rows_v,
            acc):
    cid = lax.axis_index("c")
    sid = lax.axis_index("s")
    base = cid * HALF

    @pl.loop(0, NCHUNK // NS + 1)
    def _(t):
        idx = sid + t * NS

        @pl.when(idx < NCHUNK)
        def _():
            pltpu.sync_copy(hp_hbm.at[pl.ds(base + idx * CH, CH)],
                            acc.at[pl.ds(idx * CH, CH)])

    plsc.subcore_barrier()

    eps = E // NS

    @pl.loop(0, eps // EKA)
    def _(b):
        e0 = sid * eps + b * EKA
        pltpu.sync_copy(src_hbm.at[pl.ds(e0, EKA)], sidx_v)
        pltpu.sync_copy(dst_hbm.at[pl.ds(e0, EKA)], didx_v)
        pltpu.sync_copy(hp_hbm.at[sidx_v], rows_v)

        @pl.loop(0, EKA, step=16)
        def _(i):
            d = didx_v[pl.ds(i, 16)]
            loc = d - base
            inb = (loc >= 0) & (loc < HALF)
            loc_v[pl.ds(i, 16)] = jnp.where(inb, loc, DUMMY)

        pltpu.sync_copy(rows_v, acc.at[loc_v], add=True)

    plsc.subcore_barrier()

    @pl.loop(0, NCHUNK // NS + 1)
    def _(t):
        idx = sid + t * NS

        @pl.when(idx < NCHUNK)
        def _():
            pltpu.sync_copy(acc.at[pl.ds(idx * CH, CH)],
                            agg_hbm.at[pl.ds(base + idx * CH, CH)])


def _agg(hp, src, dst):
    k = pl.kernel(
        _sc_agg,
        out_type=jax.ShapeDtypeStruct((N, H), jnp.float32),
        mesh=_vmesh,
        compiler_params=_sc_params,
        scratch_types=[
            pltpu.VMEM((EKA,), jnp.int32),
            pltpu.VMEM((EKA,), jnp.int32),
            pltpu.VMEM((EKA,), jnp.int32),
            pltpu.VMEM((EKA, H), jnp.float32),
            pltpu.VMEM_SHARED((SPROWS, H), jnp.float32),
        ],
    )
    return k(hp, src, dst)


def _sc_combine(a_hbm, b_hbm, src_hbm, dst_hbm, c_hbm, sidx_v, didx_v, arows,
                brows, crows):
    cid = lax.axis_index("c")
    sid = lax.axis_index("s")
    wid = sid * NC + cid
    epw = E // (NC * NS)

    def block(e0, k):
        pltpu.sync_copy(src_hbm.at[pl.ds(e0, k)], sidx_v.at[pl.ds(0, k)])
        pltpu.sync_copy(dst_hbm.at[pl.ds(e0, k)], didx_v.at[pl.ds(0, k)])
        pltpu.sync_copy(a_hbm.at[sidx_v.at[pl.ds(0, k)]],
                        arows.at[pl.ds(0, k)])
        pltpu.sync_copy(b_hbm.at[didx_v.at[pl.ds(0, k)]],
                        brows.at[pl.ds(0, k)])

        @pl.loop(0, k)
        def _(r):
            @pl.loop(0, H, step=16)
            def _(j):
                crows[r, pl.ds(j, 16)] = (arows[r, pl.ds(j, 16)] +
                                          brows[r, pl.ds(j, 16)])

        pltpu.sync_copy(crows.at[pl.ds(0, k)], c_hbm.at[pl.ds(e0, k)])

    @pl.loop(0, epw // EK)
    def _(b):
        block(wid * epw + b * EK, EK)

    if epw % EK:
        block(wid * epw + (epw // EK) * EK, epw % EK)


def _combine(a, b, src, dst):
    k = pl.kernel(
        _sc_combine,
        out_type=jax.ShapeDtypeStruct((E, H), jnp.float32),
        mesh=_vmesh,
        compiler_params=_sc_params,
        scratch_types=[
            pltpu.VMEM((EK,), jnp.int32),
            pltpu.VMEM((EK,), jnp.int32),
            pltpu.VMEM((EK, H), jnp.float32),
            pltpu.VMEM((EK, H), jnp.float32),
            pltpu.VMEM((EK, H), jnp.float32),
        ],
    )
    return k(a, b, src, dst)


BR = 1000
BE = 1000


def _ln(y, g, b):
    mu = jnp.mean(y, axis=-1, keepdims=True)
    var = jnp.mean((y - mu) ** 2, axis=-1, keepdims=True)
    return (y - mu) * lax.rsqrt(var + 1e-5) * g + b


def _enc_body(nf, w, b, g, beta, o):
    y = jnp.dot(nf[...], w[...], preferred_element_type=jnp.float32) + b[...]
    o[...] = jnp.maximum(_ln(y, g[...], beta[...]), 0.0)


def _encoder(nf, w, b, g, beta):
    return pl.pallas_call(
        _enc_body,
        grid=(N // BR,),
        in_specs=[
            pl.BlockSpec((BR, D_IN), lambda i: (i, 0)),
            pl.BlockSpec((D_IN, H), lambda i: (0, 0)),
            pl.BlockSpec((1, H), lambda i: (0, 0)),
            pl.BlockSpec((1, H), lambda i: (0, 0)),
            pl.BlockSpec((1, H), lambda i: (0, 0)),
        ],
        out_specs=pl.BlockSpec((BR, H), lambda i: (i, 0)),
        out_shape=jax.ShapeDtypeStruct((N, H), jnp.float32),
    )(nf, w, b, g, beta)


def _pre_body(x, w, deg, o):
    dinv = lax.rsqrt(deg[...][:, :1] + 1.0)
    o[...] = jnp.dot(x[...], w[...],
                     preferred_element_type=jnp.float32) * dinv


def _pre(x, w, deg):
    return pl.pallas_call(
        _pre_body,
        grid=(N // BR,),
        in_specs=[
            pl.BlockSpec((BR, H), lambda i: (i, 0)),
            pl.BlockSpec((H, H), lambda i: (0, 0)),
            pl.BlockSpec((BR, 16), lambda i: (i, 0)),
        ],
        out_specs=pl.BlockSpec((BR, H), lambda i: (i, 0)),
        out_shape=jax.ShapeDtypeStruct((N, H), jnp.float32),
    )(x, w, deg)


def _post_body(agg, x, deg, b, g, beta, o):
    dinv = lax.rsqrt(deg[...][:, :1] + 1.0)
    y = agg[...] * dinv + b[...]
    o[...] = jnp.maximum(_ln(y, g[...], beta[...]), 0.0) + x[...]


def _post(agg, x, deg, b, g, beta):
    return pl.pallas_call(
        _post_body,
        grid=(N // BR,),
        in_specs=[
            pl.BlockSpec((BR, H), lambda i: (i, 0)),
            pl.BlockSpec((BR, H), lambda i: (i, 0)),
            pl.BlockSpec((BR, 16), lambda i: (i, 0)),
            pl.BlockSpec((1, H), lambda i: (0, 0)),
            pl.BlockSpec((1, H), lambda i: (0, 0)),
            pl.BlockSpec((1, H), lambda i: (0, 0)),
        ],
        out_specs=pl.BlockSpec((BR, H), lambda i: (i, 0)),
        out_shape=jax.ShapeDtypeStruct((N, H), jnp.float32),
    )(agg, x, deg, b, g, beta)


def _ab_body(x, wa, wb, b1, ao, bo):
    xv = x[...]
    ao[...] = jnp.dot(xv, wa[...], preferred_element_type=jnp.float32) + b1[...]
    bo[...] = jnp.dot(xv, wb[...], preferred_element_type=jnp.float32)


def _ab(x, wa, wb, b1):
    return pl.pallas_call(
        _ab_body,
        grid=(N // BR,),
        in_specs=[
            pl.BlockSpec((BR, H), lambda i: (i, 0)),
            pl.BlockSpec((H, H), lambda i: (0, 0)),
            pl.BlockSpec((H, H), lambda i: (0, 0)),
            pl.BlockSpec((1, H), lambda i: (0, 0)),
        ],
        out_specs=[
            pl.BlockSpec((BR, H), lambda i: (i, 0)),
            pl.BlockSpec((BR, H), lambda i: (i, 0)),
        ],
        out_shape=[
            jax.ShapeDtypeStruct((N, H), jnp.float32),
            jax.ShapeDtypeStruct((N, H), jnp.float32),
        ],
    )(x, wa, wb, b1)


def _logits_body(c, w2, b2, o):
    h = jnp.maximum(c[...], 0.0)
    o[...] = jnp.sum(h * w2[...], axis=1, keepdims=True) + b2[...]


def _logits(c, w2, b2):
    return pl.pallas_call(
        _logits_body,
        grid=(E // BE,),
        in_specs=[
            pl.BlockSpec((BE, H), lambda i: (i, 0)),
            pl.BlockSpec((1, H), lambda i: (0, 0)),
            pl.BlockSpec((1, 1), lambda i: (0, 0)),
        ],
        out_specs=pl.BlockSpec((BE, 1), lambda i: (i, 0)),
        out_shape=jax.ShapeDtypeStruct((E, 1), jnp.float32),
    )(c, w2, b2)


def kernel(node_features, edge_attr, enc_W, enc_b, enc_g, enc_beta, conv_W,
           conv_b, norm_g, norm_b, mlp_W1, mlp_b1, mlp_W2, mlp_b2, edge_index):
    src = edge_index[0]
    dst = edge_index[1]
    r = lambda v: v.reshape(1, -1)

    deg = _deg(dst)
    x = _encoder(node_features, enc_W, r(enc_b), r(enc_g), r(enc_beta))

    for i in range(conv_W.shape[0]):
        hp = _pre(x, conv_W[i], deg)
        agg = _agg(hp, src, dst)
        x = _post(agg, x, deg, r(conv_b[i]), r(norm_g[i]), r(norm_b[i]))

    a, b = _ab(x, mlp_W1[:H], mlp_W1[H:], r(mlp_b1))
    c = _combine(a, b, src, dst)
    logits = _logits(c, mlp_W2.reshape(1, H), mlp_b2.reshape(1, 1))
    return (x, logits.reshape(E))


# speedup vs baseline: 8.2221x; 1.5879x over previous
import jax
import jax.numpy as jnp
from jax import lax
from jax.experimental import pallas as pl
from jax.experimental.pallas import tpu as pltpu
from jax.experimental.pallas import tpu_sc as plsc

N = 50000
E = 800000
H = 64
D_IN = 128
NC = 2
NS = 16
HALF = N // NC
DUMMY = HALF
SPROWS = HALF + 8
EKB = 128
EPS = E // NS
NBLK = EPS // EKB
TAIL = EPS % EKB
UA = 3
EPW = E // (NC * NS)
NBLKW = EPW // EKB
TAILW = EPW % EKB
UC = 3
CH = 200
NCHUNK = HALF // CH

_vmesh = plsc.VectorSubcoreMesh(core_axis_name="c", subcore_axis_name="s")
_sc_params = pltpu.CompilerParams(use_tc_tiling_on_sc=False)


def _spmem_chunks(sid, fn):
    @pl.loop(0, NCHUNK // NS + 1)
    def _(t):
        idx = sid + t * NS

        @pl.when(idx < NCHUNK)
        def _():
            fn(idx)


def _dst_local(didx_row, loc_row, base, k):
    @pl.loop(0, k, step=16)
    def _(i):
        d = didx_row[pl.ds(i, 16)]
        loc = d - base
        inb = (loc >= 0) & (loc < HALF)
        loc_row[pl.ds(i, 16)] = jnp.where(inb, loc, DUMMY)


def _sc_deg(dst_hbm, deg_hbm, zbuf, ones_v, idx_v, loc_v, tidx, tloc, acc,
            semi, semsc):
    cid = lax.axis_index("c")
    sid = lax.axis_index("s")
    base = cid * HALF

    z16 = jnp.zeros((16,), jnp.float32)

    @pl.loop(0, CH)
    def _(r):
        zbuf[r, :] = z16

    _spmem_chunks(sid, lambda idx: pltpu.sync_copy(
        zbuf, acc.at[pl.ds(idx * CH, CH)]))

    lane = lax.iota(jnp.int32, 16)
    unit = jnp.where(lane == 0, jnp.float32(1.0), jnp.float32(0.0))

    @pl.loop(0, EKB)
    def _(r):
        ones_v[r, :] = unit

    plsc.subcore_barrier()

    ebase = sid * EPS

    @pl.loop(0, NBLK // UA)
    def _(it):
        b0 = it * UA
        cps = [pltpu.async_copy(dst_hbm.at[pl.ds(ebase + (b0 + j) * EKB, EKB)],
                                idx_v.at[j], semi.at[j]) for j in range(UA)]
        ss = []
        for j in range(UA):
            cps[j].wait()
            _dst_local(idx_v.at[j], loc_v.at[j], base, EKB)
            ss.append(pltpu.async_copy(ones_v, acc.at[loc_v.at[j]],
                                       semsc.at[j], add=True))
        for s in ss:
            s.wait()

    e0 = ebase + NBLK * EKB
    pltpu.sync_copy(dst_hbm.at[pl.ds(e0, TAIL)], tidx)
    _dst_local(tidx, tloc, base, TAIL)
    pltpu.sync_copy(ones_v.at[pl.ds(0, TAIL)], acc.at[tloc], add=True)

    plsc.subcore_barrier()

    _spmem_chunks(sid, lambda idx: pltpu.sync_copy(
        acc.at[pl.ds(idx * CH, CH)],
        deg_hbm.at[pl.ds(base + idx * CH, CH)]))


def _deg(dst):
    k = pl.kernel(
        _sc_deg,
        out_type=jax.ShapeDtypeStruct((N, 16), jnp.float32),
        mesh=_vmesh,
        compiler_params=_sc_params,
        scratch_types=[
            pltpu.VMEM((CH, 16), jnp.float32),
            pltpu.VMEM((EKB, 16), jnp.float32),
            pltpu.VMEM((UA, EKB), jnp.int32),
            pltpu.VMEM((UA, EKB), jnp.int32),
            pltpu.VMEM((TAIL,), jnp.int32),
            pltpu.VMEM((TAIL,), jnp.int32),
            pltpu.VMEM_SHARED((SPROWS, 16), jnp.float32),
            pltpu.SemaphoreType.DMA((UA,)),
            pltpu.SemaphoreType.DMA((UA,)),
        ],
    )
    return k(dst)


def _sc_agg(hp_hbm, src_hbm, dst_hbm, agg_hbm, sidx, didx, rows,
            tsidx, tdidx, trows, acc, semi, semg, semsc):
    cid = lax.axis_index("c")
    sid = lax.axis_index("s")
    base = cid * HALF

    _spmem_chunks(sid, lambda idx: pltpu.sync_copy(
        hp_hbm.at[pl.ds(base + idx * CH, CH)], acc.at[pl.ds(idx * CH, CH)]))

    plsc.subcore_barrier()

    ebase = sid * EPS

    @pl.loop(0, NBLK // UA)
    def _(it):
        b0 = it * UA
        cps = []
        for j in range(UA):
            e0 = ebase + (b0 + j) * EKB
            c1 = pltpu.async_copy(src_hbm.at[pl.ds(e0, EKB)], sidx.at[j],
                                  semi.at[j])
            c2 = pltpu.async_copy(dst_hbm.at[pl.ds(e0, EKB)], didx.at[j],
                                  semi.at[j])
            cps.append((c1, c2))
        gs = []
        for j in range(UA):
            cps[j][0].wait()
            cps[j][1].wait()
            gs.append(pltpu.async_copy(hp_hbm.at[sidx.at[j]], rows.at[j],
                                       semg.at[j]))
        for j in range(UA):
            _dst_local(didx.at[j], didx.at[j], base, EKB)
        ss = []
        for j in range(UA):
            gs[j].wait()
            ss.append(pltpu.async_copy(rows.at[j], acc.at[didx.at[j]],
                                       semsc.at[j], add=True))
        for s in ss:
            s.wait()

    e0 = ebase + NBLK * EKB
    pltpu.sync_copy(src_hbm.at[pl.ds(e0, TAIL)], tsidx)
    pltpu.sync_copy(dst_hbm.at[pl.ds(e0, TAIL)], tdidx)
    pltpu.sync_copy(hp_hbm.at[tsidx], trows)
    _dst_local(tdidx, tdidx, base, TAIL)
    pltpu.sync_copy(trows, acc.at[tdidx], add=True)

    plsc.subcore_barrier()

    _spmem_chunks(sid, lambda idx: pltpu.sync_copy(
        acc.at[pl.ds(idx * CH, CH)],
        agg_hbm.at[pl.ds(base + idx * CH, CH)]))


def _agg(hp, src, dst):
    k = pl.kernel(
        _sc_agg,
        out_type=jax.ShapeDtypeStruct((N, H), jnp.float32),
        mesh=_vmesh,
        compiler_params=_sc_params,
        scratch_types=[
            pltpu.VMEM((UA, EKB), jnp.int32),
            pltpu.VMEM((UA, EKB), jnp.int32),
            pltpu.VMEM((UA, EKB, H), jnp.float32),
            pltpu.VMEM((TAIL,), jnp.int32),
            pltpu.VMEM((TAIL,), jnp.int32),
            pltpu.VMEM((TAIL, H), jnp.float32),
            pltpu.VMEM_SHARED((SPROWS, H), jnp.float32),
            pltpu.SemaphoreType.DMA((UA,)),
            pltpu.SemaphoreType.DMA((UA,)),
            pltpu.SemaphoreType.DMA((UA,)),
        ],
    )
    return k(hp, src, dst)


def _sc_combine(a_hbm, b_hbm, src_hbm, dst_hbm, c_hbm, sidx, didx,
                arows0, arows1, arows2, brows0, brows1, brows2,
                tsidx, tdidx, tarows, tbrows,
                semi, semga, semgb, semo):
    cid = lax.axis_index("c")
    sid = lax.axis_index("s")
    wid = sid * NC + cid
    ebase = wid * EPW
    arows = [arows0, arows1, arows2]
    brows = [brows0, brows1, brows2]

    def addrows(a_ref, b_ref, k):
        @pl.loop(0, k)
        def _(rr):
            for jj in range(0, H, 16):
                a_ref[rr, pl.ds(jj, 16)] = (a_ref[rr, pl.ds(jj, 16)] +
                                            b_ref[rr, pl.ds(jj, 16)])

    @pl.loop(0, NBLKW // UC)
    def _(it):
        b0 = it * UC
        cps = []
        for j in range(UC):
            e0 = ebase + (b0 + j) * EKB
            c1 = pltpu.async_copy(src_hbm.at[pl.ds(e0, EKB)], sidx.at[j],
                                  semi.at[j])
            c2 = pltpu.async_copy(dst_hbm.at[pl.ds(e0, EKB)], didx.at[j],
                                  semi.at[j])
            cps.append((c1, c2))
        gs = []
        for j in range(UC):
            cps[j][0].wait()
            cps[j][1].wait()
            ga = pltpu.async_copy(a_hbm.at[sidx.at[j]], arows[j], semga.at[j])
            gb = pltpu.async_copy(b_hbm.at[didx.at[j]], brows[j], semgb.at[j])
            gs.append((ga, gb))
        outs = []
        for j in range(UC):
            gs[j][0].wait()
            gs[j][1].wait()
            addrows(arows[j], brows[j], EKB)
            e0 = ebase + (b0 + j) * EKB
            outs.append(pltpu.async_copy(arows[j], c_hbm.at[pl.ds(e0, EKB)],
                                         semo.at[j]))
        for o in outs:
            o.wait()

    e0 = ebase + NBLKW * EKB
    pltpu.sync_copy(src_hbm.at[pl.ds(e0, TAILW)], tsidx)
    pltpu.sync_copy(dst_hbm.at[pl.ds(e0, TAILW)], tdidx)
    pltpu.sync_copy(a_hbm.at[tsidx], tarows)
    pltpu.sync_copy(b_hbm.at[tdidx], tbrows)
    addrows(tarows, tbrows, TAILW)
    pltpu.sync_copy(tarows, c_hbm.at[pl.ds(e0, TAILW)])


def _combine(a, b, src, dst):
    k = pl.kernel(
        _sc_combine,
        out_type=jax.ShapeDtypeStruct((E, H), jnp.float32),
        mesh=_vmesh,
        compiler_params=_sc_params,
        scratch_types=[
            pltpu.VMEM((UC, EKB), jnp.int32),
            pltpu.VMEM((UC, EKB), jnp.int32),
            pltpu.VMEM((EKB, H), jnp.float32),
            pltpu.VMEM((EKB, H), jnp.float32),
            pltpu.VMEM((EKB, H), jnp.float32),
            pltpu.VMEM((EKB, H), jnp.float32),
            pltpu.VMEM((EKB, H), jnp.float32),
            pltpu.VMEM((EKB, H), jnp.float32),
            pltpu.VMEM((TAILW,), jnp.int32),
            pltpu.VMEM((TAILW,), jnp.int32),
            pltpu.VMEM((TAILW, H), jnp.float32),
            pltpu.VMEM((TAILW, H), jnp.float32),
            pltpu.SemaphoreType.DMA((UC,)),
            pltpu.SemaphoreType.DMA((UC,)),
            pltpu.SemaphoreType.DMA((UC,)),
            pltpu.SemaphoreType.DMA((UC,)),
        ],
    )
    return k(a, b, src, dst)


BR = 1000
BE = 1000


def _ln(y, g, b):
    mu = jnp.mean(y, axis=-1, keepdims=True)
    var = jnp.mean((y - mu) ** 2, axis=-1, keepdims=True)
    return (y - mu) * lax.rsqrt(var + 1e-5) * g + b


def _enc_body(nf, w, b, g, beta, o):
    y = jnp.dot(nf[...], w[...], preferred_element_type=jnp.float32) + b[...]
    o[...] = jnp.maximum(_ln(y, g[...], beta[...]), 0.0)


def _encoder(nf, w, b, g, beta):
    return pl.pallas_call(
        _enc_body,
        grid=(N // BR,),
        in_specs=[
            pl.BlockSpec((BR, D_IN), lambda i: (i, 0)),
            pl.BlockSpec((D_IN, H), lambda i: (0, 0)),
            pl.BlockSpec((1, H), lambda i: (0, 0)),
            pl.BlockSpec((1, H), lambda i: (0, 0)),
            pl.BlockSpec((1, H), lambda i: (0, 0)),
        ],
        out_specs=pl.BlockSpec((BR, H), lambda i: (i, 0)),
        out_shape=jax.ShapeDtypeStruct((N, H), jnp.float32),
    )(nf, w, b, g, beta)


def _pre_body(x, w, deg, o):
    dinv = lax.rsqrt(deg[...][:, :1] + 1.0)
    o[...] = jnp.dot(x[...], w[...],
                     preferred_element_type=jnp.float32) * dinv


def _pre(x, w, deg):
    return pl.pallas_call(
        _pre_body,
        grid=(N // BR,),
        in_specs=[
            pl.BlockSpec((BR, H), lambda i: (i, 0)),
            pl.BlockSpec((H, H), lambda i: (0, 0)),
            pl.BlockSpec((BR, 16), lambda i: (i, 0)),
        ],
        out_specs=pl.BlockSpec((BR, H), lambda i: (i, 0)),
        out_shape=jax.ShapeDtypeStruct((N, H), jnp.float32),
    )(x, w, deg)


def _post_body(agg, x, deg, b, g, beta, o):
    dinv = lax.rsqrt(deg[...][:, :1] + 1.0)
    y = agg[...] * dinv + b[...]
    o[...] = jnp.maximum(_ln(y, g[...], beta[...]), 0.0) + x[...]


def _post(agg, x, deg, b, g, beta):
    return pl.pallas_call(
        _post_body,
        grid=(N // BR,),
        in_specs=[
            pl.BlockSpec((BR, H), lambda i: (i, 0)),
            pl.BlockSpec((BR, H), lambda i: (i, 0)),
            pl.BlockSpec((BR, 16), lambda i: (i, 0)),
            pl.BlockSpec((1, H), lambda i: (0, 0)),
            pl.BlockSpec((1, H), lambda i: (0, 0)),
            pl.BlockSpec((1, H), lambda i: (0, 0)),
        ],
        out_specs=pl.BlockSpec((BR, H), lambda i: (i, 0)),
        out_shape=jax.ShapeDtypeStruct((N, H), jnp.float32),
    )(agg, x, deg, b, g, beta)


def _ab_body(x, wa, wb, b1, ao, bo):
    xv = x[...]
    ao[...] = jnp.dot(xv, wa[...], preferred_element_type=jnp.float32) + b1[...]
    bo[...] = jnp.dot(xv, wb[...], preferred_element_type=jnp.float32)


def _ab(x, wa, wb, b1):
    return pl.pallas_call(
        _ab_body,
        grid=(N // BR,),
        in_specs=[
            pl.BlockSpec((BR, H), lambda i: (i, 0)),
            pl.BlockSpec((H, H), lambda i: (0, 0)),
            pl.BlockSpec((H, H), lambda i: (0, 0)),
            pl.BlockSpec((1, H), lambda i: (0, 0)),
        ],
        out_specs=[
            pl.BlockSpec((BR, H), lambda i: (i, 0)),
            pl.BlockSpec((BR, H), lambda i: (i, 0)),
        ],
        out_shape=[
            jax.ShapeDtypeStruct((N, H), jnp.float32),
            jax.ShapeDtypeStruct((N, H), jnp.float32),
        ],
    )(x, wa, wb, b1)


def _logits_body(c, w2, b2, o):
    h = jnp.maximum(c[...], 0.0)
    o[...] = jnp.sum(h * w2[...], axis=1, keepdims=True) + b2[...]


def _logits(c, w2, b2):
    return pl.pallas_call(
        _logits_body,
        grid=(E // BE,),
        in_specs=[
            pl.BlockSpec((BE, H), lambda i: (i, 0)),
            pl.BlockSpec((1, H), lambda i: (0, 0)),
            pl.BlockSpec((1, 1), lambda i: (0, 0)),
        ],
        out_specs=pl.BlockSpec((BE, 1), lambda i: (i, 0)),
        out_shape=jax.ShapeDtypeStruct((E, 1), jnp.float32),
    )(c, w2, b2)


def kernel(node_features, edge_attr, enc_W, enc_b, enc_g, enc_beta, conv_W,
           conv_b, norm_g, norm_b, mlp_W1, mlp_b1, mlp_W2, mlp_b2, edge_index):
    src = edge_index[0]
    dst = edge_index[1]
    r = lambda v: v.reshape(1, -1)

    deg = _deg(dst)
    x = _encoder(node_features, enc_W, r(enc_b), r(enc_g), r(enc_beta))

    for i in range(conv_W.shape[0]):
        hp = _pre(x, conv_W[i], deg)
        agg = _agg(hp, src, dst)
        x = _post(agg, x, deg, r(conv_b[i]), r(norm_g[i]), r(norm_b[i]))

    a, b = _ab(x, mlp_W1[:H], mlp_W1[H:], r(mlp_b1))
    c = _combine(a, b, src, dst)
    logits = _logits(c, mlp_W2.reshape(1, H), mlp_b2.reshape(1, 1))
    return (x, logits.reshape(E))


# speedup vs baseline: 11.4813x; 1.3964x over previous
import jax
import jax.numpy as jnp
from jax import lax
from jax.experimental import pallas as pl
from jax.experimental.pallas import tpu as pltpu
from jax.experimental.pallas import tpu_sc as plsc

N = 50000
E = 800000
H = 64
D_IN = 128
NC = 2
NS = 16
HALF = N // NC
DUMMY = HALF
SPROWS = HALF + 8
EKB = 128
EPS = E // NS
NBLK = EPS // EKB
TAIL = EPS % EKB
UA = 3
EPW = E // (NC * NS)
NBLKW = EPW // EKB
TAILW = EPW % EKB
UC = 3
CH = 200
NCHUNK = HALF // CH
NCHUNKD = N // CH

_vmesh = plsc.VectorSubcoreMesh(core_axis_name="c", subcore_axis_name="s")
_sc_params = pltpu.CompilerParams(use_tc_tiling_on_sc=False)


def _chunks(sid, nchunk, fn):
    @pl.loop(0, nchunk // NS + 1)
    def _(t):
        idx = sid + t * NS

        @pl.when(idx < nchunk)
        def _():
            fn(idx)


def _dst_local(didx_row, base, k):
    @pl.loop(0, k, step=16)
    def _(i):
        d = didx_row[pl.ds(i, 16)]
        loc = d - base
        inb = (loc >= 0) & (loc < HALF)
        didx_row[pl.ds(i, 16)] = jnp.where(inb, loc, DUMMY)


def _sc_deg(ei_hbm, deg_hbm, zbuf, ones_v, idx_v, tidx, acc, semi, semsc):
    cid = lax.axis_index("c")
    sid = lax.axis_index("s")

    z16 = jnp.zeros((16,), jnp.float32)

    @pl.loop(0, CH)
    def _(r):
        zbuf[r, :] = z16

    _chunks(sid, NCHUNKD, lambda idx: pltpu.sync_copy(
        zbuf, acc.at[pl.ds(idx * CH, CH)]))

    lane = lax.iota(jnp.int32, 16)
    unit = jnp.where(lane == 0, jnp.float32(1.0), jnp.float32(0.0))

    @pl.loop(0, EKB)
    def _(r):
        ones_v[r, :] = unit

    plsc.subcore_barrier()

    ebase = cid * (E // NC) + sid * EPW

    @pl.loop(0, NBLKW // UC)
    def _(it):
        b0 = it * UC
        cps = [pltpu.async_copy(
            ei_hbm.at[1, pl.ds(ebase + (b0 + j) * EKB, EKB)],
            idx_v.at[j], semi.at[j]) for j in range(UC)]
        ss = []
        for j in range(UC):
            cps[j].wait()
            ss.append(pltpu.async_copy(ones_v, acc.at[idx_v.at[j]],
                                       semsc.at[j], add=True))
        for s in ss:
            s.wait()

    e0 = ebase + NBLKW * EKB
    pltpu.sync_copy(ei_hbm.at[1, pl.ds(e0, TAILW)], tidx)
    pltpu.sync_copy(ones_v.at[pl.ds(0, TAILW)], acc.at[tidx], add=True)

    plsc.subcore_barrier()

    _chunks(sid, NCHUNKD, lambda idx: pltpu.sync_copy(
        acc.at[pl.ds(idx * CH, CH)],
        deg_hbm.at[pl.ds(cid * N + idx * CH, CH)]))


def _deg(ei):
    k = pl.kernel(
        _sc_deg,
        out_type=jax.ShapeDtypeStruct((2 * N, 16), jnp.float32),
        mesh=_vmesh,
        compiler_params=_sc_params,
        scratch_types=[
            pltpu.VMEM((CH, 16), jnp.float32),
            pltpu.VMEM((EKB, 16), jnp.float32),
            pltpu.VMEM((UC, EKB), jnp.int32),
            pltpu.VMEM((TAILW,), jnp.int32),
            pltpu.VMEM_SHARED((N, 16), jnp.float32),
            pltpu.SemaphoreType.DMA((UC,)),
            pltpu.SemaphoreType.DMA((UC,)),
        ],
    )
    return k(ei)


def _sc_agg(hp_hbm, ei_hbm, agg_hbm, sidx, didx, rows,
            tsidx, tdidx, trows, acc, semi, semg, semsc):
    cid = lax.axis_index("c")
    sid = lax.axis_index("s")
    base = cid * HALF

    _chunks(sid, NCHUNK, lambda idx: pltpu.sync_copy(
        hp_hbm.at[pl.ds(base + idx * CH, CH)], acc.at[pl.ds(idx * CH, CH)]))

    plsc.subcore_barrier()

    ebase = sid * EPS

    @pl.loop(0, NBLK // UA)
    def _(it):
        b0 = it * UA
        cps = []
        for j in range(UA):
            e0 = ebase + (b0 + j) * EKB
            c1 = pltpu.async_copy(ei_hbm.at[0, pl.ds(e0, EKB)], sidx.at[j],
                                  semi.at[j])
            c2 = pltpu.async_copy(ei_hbm.at[1, pl.ds(e0, EKB)], didx.at[j],
                                  semi.at[j])
            cps.append((c1, c2))
        gs = []
        for j in range(UA):
            cps[j][0].wait()
            cps[j][1].wait()
            gs.append(pltpu.async_copy(hp_hbm.at[sidx.at[j]], rows.at[j],
                                       semg.at[j]))
        for j in range(UA):
            _dst_local(didx.at[j], base, EKB)
        ss = []
        for j in range(UA):
            gs[j].wait()
            ss.append(pltpu.async_copy(rows.at[j], acc.at[didx.at[j]],
                                       semsc.at[j], add=True))
        for s in ss:
            s.wait()

    e0 = ebase + NBLK * EKB
    pltpu.sync_copy(ei_hbm.at[0, pl.ds(e0, TAIL)], tsidx)
    pltpu.sync_copy(ei_hbm.at[1, pl.ds(e0, TAIL)], tdidx)
    pltpu.sync_copy(hp_hbm.at[tsidx], trows)
    _dst_local(tdidx, base, TAIL)
    pltpu.sync_copy(trows, acc.at[tdidx], add=True)

    plsc.subcore_barrier()

    _chunks(sid, NCHUNK, lambda idx: pltpu.sync_copy(
        acc.at[pl.ds(idx * CH, CH)],
        agg_hbm.at[pl.ds(base + idx * CH, CH)]))


def _agg(hp, ei):
    k = pl.kernel(
        _sc_agg,
        out_type=jax.ShapeDtypeStruct((N, H), jnp.float32),
        mesh=_vmesh,
        compiler_params=_sc_params,
        scratch_types=[
            pltpu.VMEM((UA, EKB), jnp.int32),
            pltpu.VMEM((UA, EKB), jnp.int32),
            pltpu.VMEM((UA, EKB, H), jnp.float32),
            pltpu.VMEM((TAIL,), jnp.int32),
            pltpu.VMEM((TAIL,), jnp.int32),
            pltpu.VMEM((TAIL, H), jnp.float32),
            pltpu.VMEM_SHARED((SPROWS, H), jnp.float32),
            pltpu.SemaphoreType.DMA((UA,)),
            pltpu.SemaphoreType.DMA((UA,)),
            pltpu.SemaphoreType.DMA((UA,)),
        ],
    )
    return k(hp, ei)


def _sc_combine(a_hbm, b_hbm, ei_hbm, c_hbm, sidx, didx,
                arows0, arows1, arows2, brows0, brows1, brows2,
                tsidx, tdidx, tarows, tbrows,
                semi, semga, semgb, semo):
    cid = lax.axis_index("c")
    sid = lax.axis_index("s")
    wid = sid * NC + cid
    ebase = wid * EPW
    arows = [arows0, arows1, arows2]
    brows = [brows0, brows1, brows2]

    def addrows(a_ref, b_ref, k):
        @pl.loop(0, k)
        def _(rr):
            for jj in range(0, H, 16):
                a_ref[rr, pl.ds(jj, 16)] = (a_ref[rr, pl.ds(jj, 16)] +
                                            b_ref[rr, pl.ds(jj, 16)])

    @pl.loop(0, NBLKW // UC)
    def _(it):
        b0 = it * UC
        cps = []
        for j in range(UC):
            e0 = ebase + (b0 + j) * EKB
            c1 = pltpu.async_copy(ei_hbm.at[0, pl.ds(e0, EKB)], sidx.at[j],
                                  semi.at[j])
            c2 = pltpu.async_copy(ei_hbm.at[1, pl.ds(e0, EKB)], didx.at[j],
                                  semi.at[j])
            cps.append((c1, c2))
        gs = []
        for j in range(UC):
            cps[j][0].wait()
            cps[j][1].wait()
            ga = pltpu.async_copy(a_hbm.at[sidx.at[j]], arows[j], semga.at[j])
            gb = pltpu.async_copy(b_hbm.at[didx.at[j]], brows[j], semgb.at[j])
            gs.append((ga, gb))
        outs = []
        for j in range(UC):
            gs[j][0].wait()
            gs[j][1].wait()
            addrows(arows[j], brows[j], EKB)
            e0 = ebase + (b0 + j) * EKB
            outs.append(pltpu.async_copy(arows[j], c_hbm.at[pl.ds(e0, EKB)],
                                         semo.at[j]))
        for o in outs:
            o.wait()

    e0 = ebase + NBLKW * EKB
    pltpu.sync_copy(ei_hbm.at[0, pl.ds(e0, TAILW)], tsidx)
    pltpu.sync_copy(ei_hbm.at[1, pl.ds(e0, TAILW)], tdidx)
    pltpu.sync_copy(a_hbm.at[tsidx], tarows)
    pltpu.sync_copy(b_hbm.at[tdidx], tbrows)
    addrows(tarows, tbrows, TAILW)
    pltpu.sync_copy(tarows, c_hbm.at[pl.ds(e0, TAILW)])


def _combine(a, b, ei):
    k = pl.kernel(
        _sc_combine,
        out_type=jax.ShapeDtypeStruct((E, H), jnp.float32),
        mesh=_vmesh,
        compiler_params=_sc_params,
        scratch_types=[
            pltpu.VMEM((UC, EKB), jnp.int32),
            pltpu.VMEM((UC, EKB), jnp.int32),
            pltpu.VMEM((EKB, H), jnp.float32),
            pltpu.VMEM((EKB, H), jnp.float32),
            pltpu.VMEM((EKB, H), jnp.float32),
            pltpu.VMEM((EKB, H), jnp.float32),
            pltpu.VMEM((EKB, H), jnp.float32),
            pltpu.VMEM((EKB, H), jnp.float32),
            pltpu.VMEM((TAILW,), jnp.int32),
            pltpu.VMEM((TAILW,), jnp.int32),
            pltpu.VMEM((TAILW, H), jnp.float32),
            pltpu.VMEM((TAILW, H), jnp.float32),
            pltpu.SemaphoreType.DMA((UC,)),
            pltpu.SemaphoreType.DMA((UC,)),
            pltpu.SemaphoreType.DMA((UC,)),
            pltpu.SemaphoreType.DMA((UC,)),
        ],
    )
    return k(a, b, ei)


BR = 1000
NBR = N // BR
BE2 = 4000


def _ln(y, g, b):
    mu = jnp.mean(y, axis=-1, keepdims=True)
    var = jnp.mean((y - mu) ** 2, axis=-1, keepdims=True)
    return (y - mu) * lax.rsqrt(var + 1e-5) * g + b


def _dinv(d0, d1):
    return lax.rsqrt(d0[...][:, :1] + d1[...][:, :1] + 1.0)


def _deg_specs():
    return [pl.BlockSpec((BR, 16), lambda i: (i, 0)),
            pl.BlockSpec((BR, 16), lambda i: (i + NBR, 0))]


def _param_specs(n):
    return [pl.BlockSpec((1, H), lambda i: (0, 0)) for _ in range(n)]


def _encpre_body(nf, w, b, g, beta, w0, d0, d1, xo, hpo):
    y = jnp.dot(nf[...], w[...], preferred_element_type=jnp.float32) + b[...]
    x = jnp.maximum(_ln(y, g[...], beta[...]), 0.0)
    xo[...] = x
    hpo[...] = jnp.dot(x, w0[...],
                       preferred_element_type=jnp.float32) * _dinv(d0, d1)


def _encpre(nf, w, b, g, beta, w0, deg):
    return pl.pallas_call(
        _encpre_body,
        grid=(NBR,),
        in_specs=[pl.BlockSpec((BR, D_IN), lambda i: (i, 0)),
                  pl.BlockSpec((D_IN, H), lambda i: (0, 0))]
                 + _param_specs(3)
                 + [pl.BlockSpec((H, H), lambda i: (0, 0))]
                 + _deg_specs(),
        out_specs=[pl.BlockSpec((BR, H), lambda i: (i, 0)),
                   pl.BlockSpec((BR, H), lambda i: (i, 0))],
        out_shape=[jax.ShapeDtypeStruct((N, H), jnp.float32),
                   jax.ShapeDtypeStruct((N, H), jnp.float32)],
    )(nf, w, b, g, beta, w0, deg, deg)


def _postpre_body(agg, x, b, g, beta, wn, d0, d1, xo, hpo):
    dinv = _dinv(d0, d1)
    y = agg[...] * dinv + b[...]
    xn = jnp.maximum(_ln(y, g[...], beta[...]), 0.0) + x[...]
    xo[...] = xn
    hpo[...] = jnp.dot(xn, wn[...],
                       preferred_element_type=jnp.float32) * dinv


def _postpre(agg, x, b, g, beta, wn, deg):
    return pl.pallas_call(
        _postpre_body,
        grid=(NBR,),
        in_specs=[pl.BlockSpec((BR, H), lambda i: (i, 0)),
                  pl.BlockSpec((BR, H), lambda i: (i, 0))]
                 + _param_specs(3)
                 + [pl.BlockSpec((H, H), lambda i: (0, 0))]
                 + _deg_specs(),
        out_specs=[pl.BlockSpec((BR, H), lambda i: (i, 0)),
                   pl.BlockSpec((BR, H), lambda i: (i, 0))],
        out_shape=[jax.ShapeDtypeStruct((N, H), jnp.float32),
                   jax.ShapeDtypeStruct((N, H), jnp.float32)],
    )(agg, x, b, g, beta, wn, deg, deg)


def _postab_body(agg, x, b, g, beta, wa, wb, b1, d0, d1, xo, ao, bo):
    dinv = _dinv(d0, d1)
    y = agg[...] * dinv + b[...]
    xn = jnp.maximum(_ln(y, g[...], beta[...]), 0.0) + x[...]
    xo[...] = xn
    ao[...] = jnp.dot(xn, wa[...], preferred_element_type=jnp.float32) + b1[...]
    bo[...] = jnp.dot(xn, wb[...], preferred_element_type=jnp.float32)


def _postab(agg, x, b, g, beta, wa, wb, b1, deg):
    return pl.pallas_call(
        _postab_body,
        grid=(NBR,),
        in_specs=[pl.BlockSpec((BR, H), lambda i: (i, 0)),
                  pl.BlockSpec((BR, H), lambda i: (i, 0))]
                 + _param_specs(3)
                 + [pl.BlockSpec((H, H), lambda i: (0, 0)),
                    pl.BlockSpec((H, H), lambda i: (0, 0))]
                 + _param_specs(1)
                 + _deg_specs(),
        out_specs=[pl.BlockSpec((BR, H), lambda i: (i, 0)),
                   pl.BlockSpec((BR, H), lambda i: (i, 0)),
                   pl.BlockSpec((BR, H), lambda i: (i, 0))],
        out_shape=[jax.ShapeDtypeStruct((N, H), jnp.float32),
                   jax.ShapeDtypeStruct((N, H), jnp.float32),
                   jax.ShapeDtypeStruct((N, H), jnp.float32)],
    )(agg, x, b, g, beta, wa, wb, b1, deg, deg)


def _logits_body(c2, w2t, b2, o):
    y = jnp.maximum(c2[...], 0.0) * w2t[...]
    s0 = jnp.sum(y[:, :H], axis=1, keepdims=True)
    s1 = jnp.sum(y[:, H:], axis=1, keepdims=True)
    o[...] = jnp.concatenate([s0, s1], axis=1) + b2[...]


def _logits(c2, w2t, b2):
    return pl.pallas_call(
        _logits_body,
        grid=(E // 2 // BE2,),
        in_specs=[
            pl.BlockSpec((BE2, 2 * H), lambda i: (i, 0)),
            pl.BlockSpec((1, 2 * H), lambda i: (0, 0)),
            pl.BlockSpec((1, 1), lambda i: (0, 0)),
        ],
        out_specs=pl.BlockSpec((BE2, 2), lambda i: (i, 0)),
        out_shape=jax.ShapeDtypeStruct((E // 2, 2), jnp.float32),
    )(c2, w2t, b2)


def kernel(node_features, edge_attr, enc_W, enc_b, enc_g, enc_beta, conv_W,
           conv_b, norm_g, norm_b, mlp_W1, mlp_b1, mlp_W2, mlp_b2, edge_index):
    r = lambda v: v.reshape(1, -1)
    L = conv_W.shape[0]

    deg = _deg(edge_index)
    x, hp = _encpre(node_features, enc_W, r(enc_b), r(enc_g), r(enc_beta),
                    conv_W[0], deg)

    for i in range(L):
        agg = _agg(hp, edge_index)
        if i < L - 1:
            x, hp = _postpre(agg, x, r(conv_b[i]), r(norm_g[i]),
                             r(norm_b[i]), conv_W[i + 1], deg)
        else:
            x, a, b = _postab(agg, x, r(conv_b[i]), r(norm_g[i]),
                              r(norm_b[i]), mlp_W1[:H], mlp_W1[H:],
                              r(mlp_b1), deg)

    c = _combine(a, b, edge_index)
    w2t = jnp.concatenate([mlp_W2, mlp_W2]).reshape(1, 2 * H)
    logits2 = _logits(c.reshape(E // 2, 2 * H), w2t, mlp_b2.reshape(1, 1))
    return (x, logits2.reshape(E))


# speedup vs baseline: 16.2088x; 1.4118x over previous
import jax
import jax.numpy as jnp
from jax import lax
from jax.experimental import pallas as pl
from jax.experimental.pallas import tpu as pltpu
from jax.experimental.pallas import tpu_sc as plsc

N = 50000
E = 800000
H = 64
D_IN = 128
NC = 2
NS = 16
HH = H // 2
EKB = 128
EPS = E // NS
NBLK = EPS // EKB
TAIL = EPS % EKB
UA = 5
EPW = E // (NC * NS)
NBLKW = EPW // EKB
TAILW = EPW % EKB
UC = 3
CH = 200
NCHUNKD = N // CH

_vmesh = plsc.VectorSubcoreMesh(core_axis_name="c", subcore_axis_name="s")
_sc_params = pltpu.CompilerParams(use_tc_tiling_on_sc=False)


def _chunks(sid, nchunk, fn):
    @pl.loop(0, nchunk // NS + 1)
    def _(t):
        idx = sid + t * NS

        @pl.when(idx < nchunk)
        def _():
            fn(idx)


def _shift(idx_row, off, k):
    @pl.loop(0, k, step=16)
    def _(i):
        idx_row[pl.ds(i, 16)] = idx_row[pl.ds(i, 16)] + off


def _sc_deg(ei_hbm, deg_hbm, zbuf, ones_v, idx_v, tidx, acc, semi, semsc):
    cid = lax.axis_index("c")
    sid = lax.axis_index("s")

    z16 = jnp.zeros((16,), jnp.float32)

    @pl.loop(0, CH)
    def _(r):
        zbuf[r, :] = z16

    _chunks(sid, NCHUNKD, lambda idx: pltpu.sync_copy(
        zbuf, acc.at[pl.ds(idx * CH, CH)]))

    lane = lax.iota(jnp.int32, 16)
    unit = jnp.where(lane == 0, jnp.float32(1.0), jnp.float32(0.0))

    @pl.loop(0, EKB)
    def _(r):
        ones_v[r, :] = unit

    plsc.subcore_barrier()

    ebase = cid * (E // NC) + sid * EPW

    @pl.loop(0, NBLKW // UC)
    def _(it):
        b0 = it * UC
        cps = [pltpu.async_copy(
            ei_hbm.at[1, pl.ds(ebase + (b0 + j) * EKB, EKB)],
            idx_v.at[j], semi.at[j]) for j in range(UC)]
        ss = []
        for j in range(UC):
            cps[j].wait()
            ss.append(pltpu.async_copy(ones_v, acc.at[idx_v.at[j]],
                                       semsc.at[j], add=True))
        for s in ss:
            s.wait()

    e0 = ebase + NBLKW * EKB
    pltpu.sync_copy(ei_hbm.at[1, pl.ds(e0, TAILW)], tidx)
    pltpu.sync_copy(ones_v.at[pl.ds(0, TAILW)], acc.at[tidx], add=True)

    plsc.subcore_barrier()

    _chunks(sid, NCHUNKD, lambda idx: pltpu.sync_copy(
        acc.at[pl.ds(idx * CH, CH)],
        deg_hbm.at[pl.ds(cid * N + idx * CH, CH)]))


def _deg(ei):
    k = pl.kernel(
        _sc_deg,
        out_type=jax.ShapeDtypeStruct((2 * N, 16), jnp.float32),
        mesh=_vmesh,
        compiler_params=_sc_params,
        scratch_types=[
            pltpu.VMEM((CH, 16), jnp.float32),
            pltpu.VMEM((EKB, 16), jnp.float32),
            pltpu.VMEM((UC, EKB), jnp.int32),
            pltpu.VMEM((TAILW,), jnp.int32),
            pltpu.VMEM_SHARED((N, 16), jnp.float32),
            pltpu.SemaphoreType.DMA((UC,)),
            pltpu.SemaphoreType.DMA((UC,)),
        ],
    )
    return k(ei)


def _sc_agg(hpl_hbm, hpr_hbm, ei_hbm, agg_hbm, sidx, didx, rows,
            tsidx, tdidx, trows, acc, semi, semg, semsc):
    cid = lax.axis_index("c")
    sid = lax.axis_index("s")
    rbase = cid * N

    def work(tab):
        _chunks(sid, NCHUNKD, lambda idx: pltpu.sync_copy(
            tab.at[pl.ds(idx * CH, CH)], acc.at[pl.ds(idx * CH, CH)]))

        plsc.subcore_barrier()

        ebase = sid * EPS

        @pl.loop(0, NBLK // UA)
        def _(it):
            b0 = it * UA
            cps = []
            for j in range(UA):
                e0 = ebase + (b0 + j) * EKB
                c1 = pltpu.async_copy(ei_hbm.at[0, pl.ds(e0, EKB)],
                                      sidx.at[j], semi.at[j])
                c2 = pltpu.async_copy(ei_hbm.at[1, pl.ds(e0, EKB)],
                                      didx.at[j], semi.at[j])
                cps.append((c1, c2))
            gs = []
            for j in range(UA):
                cps[j][0].wait()
                gs.append(pltpu.async_copy(tab.at[sidx.at[j]], rows.at[j],
                                           semg.at[j]))
            ss = []
            for j in range(UA):
                cps[j][1].wait()
                gs[j].wait()
                ss.append(pltpu.async_copy(rows.at[j], acc.at[didx.at[j]],
                                           semsc.at[j], add=True))
            for s in ss:
                s.wait()

        e0 = ebase + NBLK * EKB
        pltpu.sync_copy(ei_hbm.at[0, pl.ds(e0, TAIL)], tsidx)
        pltpu.sync_copy(ei_hbm.at[1, pl.ds(e0, TAIL)], tdidx)
        pltpu.sync_copy(tab.at[tsidx], trows)
        pltpu.sync_copy(trows, acc.at[tdidx], add=True)

    @pl.when(cid == 0)
    def _():
        work(hpl_hbm)

    @pl.when(cid == 1)
    def _():
        work(hpr_hbm)

    plsc.subcore_barrier()

    _chunks(sid, NCHUNKD, lambda idx: pltpu.sync_copy(
        acc.at[pl.ds(idx * CH, CH)],
        agg_hbm.at[pl.ds(rbase + idx * CH, CH)]))


def _agg(hpl, hpr, ei):
    k = pl.kernel(
        _sc_agg,
        out_type=jax.ShapeDtypeStruct((2 * N, HH), jnp.float32),
        mesh=_vmesh,
        compiler_params=_sc_params,
        scratch_types=[
            pltpu.VMEM((UA, EKB), jnp.int32),
            pltpu.VMEM((UA, EKB), jnp.int32),
            pltpu.VMEM((UA, EKB, HH), jnp.float32),
            pltpu.VMEM((TAIL,), jnp.int32),
            pltpu.VMEM((TAIL,), jnp.int32),
            pltpu.VMEM((TAIL, HH), jnp.float32),
            pltpu.VMEM_SHARED((N, HH), jnp.float32),
            pltpu.SemaphoreType.DMA((UA,)),
            pltpu.SemaphoreType.DMA((UA,)),
            pltpu.SemaphoreType.DMA((UA,)),
        ],
    )
    return k(hpl, hpr, ei)


def _sc_combine(a_hbm, b_hbm, ei_hbm, c_hbm, sidx, didx,
                arows0, arows1, arows2, brows0, brows1, brows2,
                crows0, crows1, crows2,
                tsidx, tdidx, tarows, tbrows, tcrows,
                semi, semga, semgb, semo):
    cid = lax.axis_index("c")
    sid = lax.axis_index("s")
    wid = sid * NC + cid
    ebase = wid * EPW
    arows = [arows0, arows1, arows2]
    brows = [brows0, brows1, brows2]
    crows = [crows0, crows1, crows2]

    def addpack(a_ref, b_ref, c_ref, k):
        @pl.loop(0, k // 2)
        def _(r2):
            for half in range(2):
                for jj in range(0, H, 16):
                    c_ref[r2, pl.ds(half * H + jj, 16)] = (
                        a_ref[2 * r2 + half, pl.ds(jj, 16)] +
                        b_ref[2 * r2 + half, pl.ds(jj, 16)])

    @pl.loop(0, NBLKW // UC)
    def _(it):
        b0 = it * UC
        cps = []
        for j in range(UC):
            e0 = ebase + (b0 + j) * EKB
            c1 = pltpu.async_copy(ei_hbm.at[0, pl.ds(e0, EKB)], sidx.at[j],
                                  semi.at[j])
            c2 = pltpu.async_copy(ei_hbm.at[1, pl.ds(e0, EKB)], didx.at[j],
                                  semi.at[j])
            cps.append((c1, c2))
        gs = []
        for j in range(UC):
            cps[j][0].wait()
            cps[j][1].wait()
            ga = pltpu.async_copy(a_hbm.at[sidx.at[j]], arows[j], semga.at[j])
            gb = pltpu.async_copy(b_hbm.at[didx.at[j]], brows[j], semgb.at[j])
            gs.append((ga, gb))
        outs = []
        for j in range(UC):
            gs[j][0].wait()
            gs[j][1].wait()
            addpack(arows[j], brows[j], crows[j], EKB)
            r0 = (ebase + (b0 + j) * EKB) // 2
            outs.append(pltpu.async_copy(crows[j],
                                         c_hbm.at[pl.ds(r0, EKB // 2)],
                                         semo.at[j]))
        for o in outs:
            o.wait()

    e0 = ebase + NBLKW * EKB
    pltpu.sync_copy(ei_hbm.at[0, pl.ds(e0, TAILW)], tsidx)
    pltpu.sync_copy(ei_hbm.at[1, pl.ds(e0, TAILW)], tdidx)
    pltpu.sync_copy(a_hbm.at[tsidx], tarows)
    pltpu.sync_copy(b_hbm.at[tdidx], tbrows)
    addpack(tarows, tbrows, tcrows, TAILW)
    pltpu.sync_copy(tcrows, c_hbm.at[pl.ds(e0 // 2, TAILW // 2)])


def _combine(a, b, ei):
    k = pl.kernel(
        _sc_combine,
        out_type=jax.ShapeDtypeStruct((E // 2, 2 * H), jnp.float32),
        mesh=_vmesh,
        compiler_params=_sc_params,
        scratch_types=[
            pltpu.VMEM((UC, EKB), jnp.int32),
            pltpu.VMEM((UC, EKB), jnp.int32),
            pltpu.VMEM((EKB, H), jnp.float32),
            pltpu.VMEM((EKB, H), jnp.float32),
            pltpu.VMEM((EKB, H), jnp.float32),
            pltpu.VMEM((EKB, H), jnp.float32),
            pltpu.VMEM((EKB, H), jnp.float32),
            pltpu.VMEM((EKB, H), jnp.float32),
            pltpu.VMEM((EKB // 2, 2 * H), jnp.float32),
            pltpu.VMEM((EKB // 2, 2 * H), jnp.float32),
            pltpu.VMEM((EKB // 2, 2 * H), jnp.float32),
            pltpu.VMEM((TAILW,), jnp.int32),
            pltpu.VMEM((TAILW,), jnp.int32),
            pltpu.VMEM((TAILW, H), jnp.float32),
            pltpu.VMEM((TAILW, H), jnp.float32),
            pltpu.VMEM((TAILW // 2, 2 * H), jnp.float32),
            pltpu.SemaphoreType.DMA((UC,)),
            pltpu.SemaphoreType.DMA((UC,)),
            pltpu.SemaphoreType.DMA((UC,)),
            pltpu.SemaphoreType.DMA((UC,)),
        ],
    )
    return k(a, b, ei)


BR = 1000
NBR = N // BR
BE2 = 4000


def _ln(y, g, b):
    mu = jnp.mean(y, axis=-1, keepdims=True)
    var = jnp.mean((y - mu) ** 2, axis=-1, keepdims=True)
    return (y - mu) * lax.rsqrt(var + 1e-5) * g + b


def _dinv(d0, d1):
    return lax.rsqrt(d0[...][:, :1] + d1[...][:, :1] + 1.0)


def _deg_specs():
    return [pl.BlockSpec((BR, 16), lambda i: (i, 0)),
            pl.BlockSpec((BR, 16), lambda i: (i + NBR, 0))]


def _half_specs():
    return [pl.BlockSpec((BR, HH), lambda i: (i, 0)),
            pl.BlockSpec((BR, HH), lambda i: (i + NBR, 0))]


def _param_specs(n):
    return [pl.BlockSpec((1, H), lambda i: (0, 0)) for _ in range(n)]


def _encpre_body(nf, w, b, g, beta, w0, d0, d1, xo, hpl, hpr):
    y = jnp.dot(nf[...], w[...], preferred_element_type=jnp.float32) + b[...]
    x = jnp.maximum(_ln(y, g[...], beta[...]), 0.0)
    xo[...] = x
    hp = jnp.dot(x, w0[...],
                 preferred_element_type=jnp.float32) * _dinv(d0, d1)
    hpl[...] = hp[:, :HH]
    hpr[...] = hp[:, HH:]


def _encpre(nf, w, b, g, beta, w0, deg):
    return pl.pallas_call(
        _encpre_body,
        grid=(NBR,),
        in_specs=[pl.BlockSpec((BR, D_IN), lambda i: (i, 0)),
                  pl.BlockSpec((D_IN, H), lambda i: (0, 0))]
                 + _param_specs(3)
                 + [pl.BlockSpec((H, H), lambda i: (0, 0))]
                 + _deg_specs(),
        out_specs=[pl.BlockSpec((BR, H), lambda i: (i, 0)),
                   pl.BlockSpec((BR, HH), lambda i: (i, 0)),
                   pl.BlockSpec((BR, HH), lambda i: (i, 0))],
        out_shape=[jax.ShapeDtypeStruct((N, H), jnp.float32),
                   jax.ShapeDtypeStruct((N, HH), jnp.float32),
                   jax.ShapeDtypeStruct((N, HH), jnp.float32)],
    )(nf, w, b, g, beta, w0, deg, deg)


def _postpre_body(al, ar, x, b, g, beta, wn, d0, d1, xo, hpl, hpr):
    dinv = _dinv(d0, d1)
    agg = jnp.concatenate([al[...], ar[...]], axis=1)
    y = agg * dinv + b[...]
    xn = jnp.maximum(_ln(y, g[...], beta[...]), 0.0) + x[...]
    xo[...] = xn
    hp = jnp.dot(xn, wn[...], preferred_element_type=jnp.float32) * dinv
    hpl[...] = hp[:, :HH]
    hpr[...] = hp[:, HH:]


def _postpre(agg2, x, b, g, beta, wn, deg):
    return pl.pallas_call(
        _postpre_body,
        grid=(NBR,),
        in_specs=_half_specs()
                 + [pl.BlockSpec((BR, H), lambda i: (i, 0))]
                 + _param_specs(3)
                 + [pl.BlockSpec((H, H), lambda i: (0, 0))]
                 + _deg_specs(),
        out_specs=[pl.BlockSpec((BR, H), lambda i: (i, 0)),
                   pl.BlockSpec((BR, HH), lambda i: (i, 0)),
                   pl.BlockSpec((BR, HH), lambda i: (i, 0))],
        out_shape=[jax.ShapeDtypeStruct((N, H), jnp.float32),
                   jax.ShapeDtypeStruct((N, HH), jnp.float32),
                   jax.ShapeDtypeStruct((N, HH), jnp.float32)],
    )(agg2, agg2, x, b, g, beta, wn, deg, deg)


def _postab_body(al, ar, x, b, g, beta, wa, wb, b1, d0, d1, xo, ao, bo):
    dinv = _dinv(d0, d1)
    agg = jnp.concatenate([al[...], ar[...]], axis=1)
    y = agg * dinv + b[...]
    xn = jnp.maximum(_ln(y, g[...], beta[...]), 0.0) + x[...]
    xo[...] = xn
    ao[...] = jnp.dot(xn, wa[...], preferred_element_type=jnp.float32) + b1[...]
    bo[...] = jnp.dot(xn, wb[...], preferred_element_type=jnp.float32)


def _postab(agg2, x, b, g, beta, wa, wb, b1, deg):
    return pl.pallas_call(
        _postab_body,
        grid=(NBR,),
        in_specs=_half_specs()
                 + [pl.BlockSpec((BR, H), lambda i: (i, 0))]
                 + _param_specs(3)
                 + [pl.BlockSpec((H, H), lambda i: (0, 0)),
                    pl.BlockSpec((H, H), lambda i: (0, 0))]
                 + _param_specs(1)
                 + _deg_specs(),
        out_specs=[pl.BlockSpec((BR, H), lambda i: (i, 0)),
                   pl.BlockSpec((BR, H), lambda i: (i, 0)),
                   pl.BlockSpec((BR, H), lambda i: (i, 0))],
        out_shape=[jax.ShapeDtypeStruct((N, H), jnp.float32),
                   jax.ShapeDtypeStruct((N, H), jnp.float32),
                   jax.ShapeDtypeStruct((N, H), jnp.float32)],
    )(agg2, agg2, x, b, g, beta, wa, wb, b1, deg, deg)


def _logits_body(c2, w2t, b2, o):
    y = jnp.maximum(c2[...], 0.0) * w2t[...]
    s0 = jnp.sum(y[:, :H], axis=1, keepdims=True)
    s1 = jnp.sum(y[:, H:], axis=1, keepdims=True)
    o[...] = jnp.concatenate([s0, s1], axis=1) + b2[...]


def _logits(c2, w2t, b2):
    return pl.pallas_call(
        _logits_body,
        grid=(E // 2 // BE2,),
        in_specs=[
            pl.BlockSpec((BE2, 2 * H), lambda i: (i, 0)),
            pl.BlockSpec((1, 2 * H), lambda i: (0, 0)),
            pl.BlockSpec((1, 1), lambda i: (0, 0)),
        ],
        out_specs=pl.BlockSpec((BE2, 2), lambda i: (i, 0)),
        out_shape=jax.ShapeDtypeStruct((E // 2, 2), jnp.float32),
    )(c2, w2t, b2)


def kernel(node_features, edge_attr, enc_W, enc_b, enc_g, enc_beta, conv_W,
           conv_b, norm_g, norm_b, mlp_W1, mlp_b1, mlp_W2, mlp_b2, edge_index):
    r = lambda v: v.reshape(1, -1)
    L = conv_W.shape[0]

    deg = _deg(edge_index)
    x, hpl, hpr = _encpre(node_features, enc_W, r(enc_b), r(enc_g),
                          r(enc_beta), conv_W[0], deg)

    for i in range(L):
        agg2 = _agg(hpl, hpr, edge_index)
        if i < L - 1:
            x, hpl, hpr = _postpre(agg2, x, r(conv_b[i]), r(norm_g[i]),
                                   r(norm_b[i]), conv_W[i + 1], deg)
        else:
            x, a, b = _postab(agg2, x, r(conv_b[i]), r(norm_g[i]),
                              r(norm_b[i]), mlp_W1[:H], mlp_W1[H:],
                              r(mlp_b1), deg)

    c2 = _combine(a, b, edge_index)
    w2t = jnp.concatenate([mlp_W2, mlp_W2]).reshape(1, 2 * H)
    logits2 = _logits(c2, w2t, mlp_b2.reshape(1, 1))
    return (x, logits2.reshape(E))


# speedup vs baseline: 16.9645x; 1.0466x over previous
import jax
import jax.numpy as jnp
from jax import lax
from jax.experimental import pallas as pl
from jax.experimental.pallas import tpu as pltpu
from jax.experimental.pallas import tpu_sc as plsc

N = 50000
E = 800000
H = 64
D_IN = 128
NC = 2
NS = 16
HH = H // 2
EKB = 128
EPS = E // NS
NBLK = EPS // EKB
TAIL = EPS % EKB
UA = 6
EPW = E // (NC * NS)
NBLKW = EPW // EKB
TAILW = EPW % EKB
UC = 3
CH = 200
NCHUNKD = N // CH

_vmesh = plsc.VectorSubcoreMesh(core_axis_name="c", subcore_axis_name="s")
_sc_params = pltpu.CompilerParams(use_tc_tiling_on_sc=False)


def _chunks(sid, nchunk, fn):
    @pl.loop(0, nchunk // NS + 1)
    def _(t):
        idx = sid + t * NS

        @pl.when(idx < nchunk)
        def _():
            fn(idx)


def _shift(idx_row, off, k):
    @pl.loop(0, k, step=16)
    def _(i):
        idx_row[pl.ds(i, 16)] = idx_row[pl.ds(i, 16)] + off


def _sc_deg(ei_hbm, deg_hbm, zbuf, ones_v, idx_v, tidx, acc, semi, semsc):
    cid = lax.axis_index("c")
    sid = lax.axis_index("s")

    z16 = jnp.zeros((16,), jnp.float32)

    @pl.loop(0, CH)
    def _(r):
        zbuf[r, :] = z16

    _chunks(sid, NCHUNKD, lambda idx: pltpu.sync_copy(
        zbuf, acc.at[pl.ds(idx * CH, CH)]))

    lane = lax.iota(jnp.int32, 16)
    unit = jnp.where(lane == 0, jnp.float32(1.0), jnp.float32(0.0))

    @pl.loop(0, EKB)
    def _(r):
        ones_v[r, :] = unit

    plsc.subcore_barrier()

    ebase = cid * (E // NC) + sid * EPW

    @pl.loop(0, NBLKW // UC)
    def _(it):
        b0 = it * UC
        cps = [pltpu.async_copy(
            ei_hbm.at[1, pl.ds(ebase + (b0 + j) * EKB, EKB)],
            idx_v.at[j], semi.at[j]) for j in range(UC)]
        ss = []
        for j in range(UC):
            cps[j].wait()
            ss.append(pltpu.async_copy(ones_v, acc.at[idx_v.at[j]],
                                       semsc.at[j], add=True))
        for s in ss:
            s.wait()

    e0 = ebase + NBLKW * EKB
    pltpu.sync_copy(ei_hbm.at[1, pl.ds(e0, TAILW)], tidx)
    pltpu.sync_copy(ones_v.at[pl.ds(0, TAILW)], acc.at[tidx], add=True)

    plsc.subcore_barrier()

    _chunks(sid, NCHUNKD, lambda idx: pltpu.sync_copy(
        acc.at[pl.ds(idx * CH, CH)],
        deg_hbm.at[pl.ds(cid * N + idx * CH, CH)]))


def _deg(ei):
    k = pl.kernel(
        _sc_deg,
        out_type=jax.ShapeDtypeStruct((2 * N, 16), jnp.float32),
        mesh=_vmesh,
        compiler_params=_sc_params,
        scratch_types=[
            pltpu.VMEM((CH, 16), jnp.float32),
            pltpu.VMEM((EKB, 16), jnp.float32),
            pltpu.VMEM((UC, EKB), jnp.int32),
            pltpu.VMEM((TAILW,), jnp.int32),
            pltpu.VMEM_SHARED((N, 16), jnp.float32),
            pltpu.SemaphoreType.DMA((UC,)),
            pltpu.SemaphoreType.DMA((UC,)),
        ],
    )
    return k(ei)


def _sc_agg(hpl_hbm, hpr_hbm, ei_hbm, agg_hbm, sidx, didx, rows,
            tsidx, tdidx, trows, acc, semi, semg, semsc):
    cid = lax.axis_index("c")
    sid = lax.axis_index("s")
    rbase = cid * N

    def work(tab):
        _chunks(sid, NCHUNKD, lambda idx: pltpu.sync_copy(
            tab.at[pl.ds(idx * CH, CH)], acc.at[pl.ds(idx * CH, CH)]))

        plsc.subcore_barrier()

        ebase = sid * EPS

        @pl.loop(0, NBLK // UA)
        def _(it):
            b0 = it * UA
            cps = []
            for j in range(UA):
                e0 = ebase + (b0 + j) * EKB
                c1 = pltpu.async_copy(ei_hbm.at[0, pl.ds(e0, EKB)],
                                      sidx.at[j], semi.at[j])
                c2 = pltpu.async_copy(ei_hbm.at[1, pl.ds(e0, EKB)],
                                      didx.at[j], semi.at[j])
                cps.append((c1, c2))
            gs = []
            for j in range(UA):
                cps[j][0].wait()
                gs.append(pltpu.async_copy(tab.at[sidx.at[j]], rows.at[j],
                                           semg.at[j]))
            ss = []
            for j in range(UA):
                cps[j][1].wait()
                gs[j].wait()
                ss.append(pltpu.async_copy(rows.at[j], acc.at[didx.at[j]],
                                           semsc.at[j], add=True))
            for s in ss:
                s.wait()

        e0 = ebase + NBLK * EKB
        pltpu.sync_copy(ei_hbm.at[0, pl.ds(e0, TAIL)], tsidx)
        pltpu.sync_copy(ei_hbm.at[1, pl.ds(e0, TAIL)], tdidx)
        pltpu.sync_copy(tab.at[tsidx], trows)
        pltpu.sync_copy(trows, acc.at[tdidx], add=True)

    @pl.when(cid == 0)
    def _():
        work(hpl_hbm)

    @pl.when(cid == 1)
    def _():
        work(hpr_hbm)

    plsc.subcore_barrier()

    _chunks(sid, NCHUNKD, lambda idx: pltpu.sync_copy(
        acc.at[pl.ds(idx * CH, CH)],
        agg_hbm.at[pl.ds(rbase + idx * CH, CH)]))


def _agg(hpl, hpr, ei):
    k = pl.kernel(
        _sc_agg,
        out_type=jax.ShapeDtypeStruct((2 * N, HH), jnp.float32),
        mesh=_vmesh,
        compiler_params=_sc_params,
        scratch_types=[
            pltpu.VMEM((UA, EKB), jnp.int32),
            pltpu.VMEM((UA, EKB), jnp.int32),
            pltpu.VMEM((UA, EKB, HH), jnp.float32),
            pltpu.VMEM((TAIL,), jnp.int32),
            pltpu.VMEM((TAIL,), jnp.int32),
            pltpu.VMEM((TAIL, HH), jnp.float32),
            pltpu.VMEM_SHARED((N, HH), jnp.float32),
            pltpu.SemaphoreType.DMA((UA,)),
            pltpu.SemaphoreType.DMA((UA,)),
            pltpu.SemaphoreType.DMA((UA,)),
        ],
    )
    return k(hpl, hpr, ei)


def _sc_combine(a_hbm, b_hbm, ei_hbm, c_hbm, sidx, didx,
                arows0, arows1, arows2, brows0, brows1, brows2,
                crows0, crows1, crows2,
                tsidx, tdidx, tarows, tbrows, tcrows,
                semi, semga, semgb, semo):
    cid = lax.axis_index("c")
    sid = lax.axis_index("s")
    wid = sid * NC + cid
    ebase = wid * EPW
    arows = [arows0, arows1, arows2]
    brows = [brows0, brows1, brows2]
    crows = [crows0, crows1, crows2]

    def addpack(a_ref, b_ref, c_ref, k):
        @pl.loop(0, k // 2)
        def _(r2):
            for half in range(2):
                for jj in range(0, H, 16):
                    c_ref[r2, pl.ds(half * H + jj, 16)] = (
                        a_ref[2 * r2 + half, pl.ds(jj, 16)] +
                        b_ref[2 * r2 + half, pl.ds(jj, 16)])

    @pl.loop(0, NBLKW // UC)
    def _(it):
        b0 = it * UC
        cps = []
        for j in range(UC):
            e0 = ebase + (b0 + j) * EKB
            c1 = pltpu.async_copy(ei_hbm.at[0, pl.ds(e0, EKB)], sidx.at[j],
                                  semi.at[j])
            c2 = pltpu.async_copy(ei_hbm.at[1, pl.ds(e0, EKB)], didx.at[j],
                                  semi.at[j])
            cps.append((c1, c2))
        gs = []
        for j in range(UC):
            cps[j][0].wait()
            cps[j][1].wait()
            ga = pltpu.async_copy(a_hbm.at[sidx.at[j]], arows[j], semga.at[j])
            gb = pltpu.async_copy(b_hbm.at[didx.at[j]], brows[j], semgb.at[j])
            gs.append((ga, gb))
        outs = []
        for j in range(UC):
            gs[j][0].wait()
            gs[j][1].wait()
            addpack(arows[j], brows[j], crows[j], EKB)
            r0 = (ebase + (b0 + j) * EKB) // 2
            outs.append(pltpu.async_copy(crows[j],
                                         c_hbm.at[pl.ds(r0, EKB // 2)],
                                         semo.at[j]))
        for o in outs:
            o.wait()

    e0 = ebase + NBLKW * EKB
    pltpu.sync_copy(ei_hbm.at[0, pl.ds(e0, TAILW)], tsidx)
    pltpu.sync_copy(ei_hbm.at[1, pl.ds(e0, TAILW)], tdidx)
    pltpu.sync_copy(a_hbm.at[tsidx], tarows)
    pltpu.sync_copy(b_hbm.at[tdidx], tbrows)
    addpack(tarows, tbrows, tcrows, TAILW)
    pltpu.sync_copy(tcrows, c_hbm.at[pl.ds(e0 // 2, TAILW // 2)])


def _combine(a, b, ei):
    k = pl.kernel(
        _sc_combine,
        out_type=jax.ShapeDtypeStruct((E // 2, 2 * H), jnp.float32),
        mesh=_vmesh,
        compiler_params=_sc_params,
        scratch_types=[
            pltpu.VMEM((UC, EKB), jnp.int32),
            pltpu.VMEM((UC, EKB), jnp.int32),
            pltpu.VMEM((EKB, H), jnp.float32),
            pltpu.VMEM((EKB, H), jnp.float32),
            pltpu.VMEM((EKB, H), jnp.float32),
            pltpu.VMEM((EKB, H), jnp.float32),
            pltpu.VMEM((EKB, H), jnp.float32),
            pltpu.VMEM((EKB, H), jnp.float32),
            pltpu.VMEM((EKB // 2, 2 * H), jnp.float32),
            pltpu.VMEM((EKB // 2, 2 * H), jnp.float32),
            pltpu.VMEM((EKB // 2, 2 * H), jnp.float32),
            pltpu.VMEM((TAILW,), jnp.int32),
            pltpu.VMEM((TAILW,), jnp.int32),
            pltpu.VMEM((TAILW, H), jnp.float32),
            pltpu.VMEM((TAILW, H), jnp.float32),
            pltpu.VMEM((TAILW // 2, 2 * H), jnp.float32),
            pltpu.SemaphoreType.DMA((UC,)),
            pltpu.SemaphoreType.DMA((UC,)),
            pltpu.SemaphoreType.DMA((UC,)),
            pltpu.SemaphoreType.DMA((UC,)),
        ],
    )
    return k(a, b, ei)


BR = 1000
NBR = N // BR
BE2 = 4000


def _ln(y, g, b):
    mu = jnp.mean(y, axis=-1, keepdims=True)
    var = jnp.mean((y - mu) ** 2, axis=-1, keepdims=True)
    return (y - mu) * lax.rsqrt(var + 1e-5) * g + b


def _dinv(d0, d1):
    return lax.rsqrt(d0[...][:, :1] + d1[...][:, :1] + 1.0)


def _deg_specs():
    return [pl.BlockSpec((BR, 16), lambda i: (i, 0)),
            pl.BlockSpec((BR, 16), lambda i: (i + NBR, 0))]


def _half_specs():
    return [pl.BlockSpec((BR, HH), lambda i: (i, 0)),
            pl.BlockSpec((BR, HH), lambda i: (i + NBR, 0))]


def _param_specs(n):
    return [pl.BlockSpec((1, H), lambda i: (0, 0)) for _ in range(n)]


def _encpre_body(nf, w, b, g, beta, w0, d0, d1, xo, hpl, hpr):
    y = jnp.dot(nf[...], w[...], preferred_element_type=jnp.float32) + b[...]
    x = jnp.maximum(_ln(y, g[...], beta[...]), 0.0)
    xo[...] = x
    hp = jnp.dot(x, w0[...],
                 preferred_element_type=jnp.float32) * _dinv(d0, d1)
    hpl[...] = hp[:, :HH]
    hpr[...] = hp[:, HH:]


def _encpre(nf, w, b, g, beta, w0, deg):
    return pl.pallas_call(
        _encpre_body,
        grid=(NBR,),
        in_specs=[pl.BlockSpec((BR, D_IN), lambda i: (i, 0)),
                  pl.BlockSpec((D_IN, H), lambda i: (0, 0))]
                 + _param_specs(3)
                 + [pl.BlockSpec((H, H), lambda i: (0, 0))]
                 + _deg_specs(),
        out_specs=[pl.BlockSpec((BR, H), lambda i: (i, 0)),
                   pl.BlockSpec((BR, HH), lambda i: (i, 0)),
                   pl.BlockSpec((BR, HH), lambda i: (i, 0))],
        out_shape=[jax.ShapeDtypeStruct((N, H), jnp.float32),
                   jax.ShapeDtypeStruct((N, HH), jnp.float32),
                   jax.ShapeDtypeStruct((N, HH), jnp.float32)],
    )(nf, w, b, g, beta, w0, deg, deg)


def _postpre_body(al, ar, x, b, g, beta, wn, d0, d1, xo, hpl, hpr):
    dinv = _dinv(d0, d1)
    agg = jnp.concatenate([al[...], ar[...]], axis=1)
    y = agg * dinv + b[...]
    xn = jnp.maximum(_ln(y, g[...], beta[...]), 0.0) + x[...]
    xo[...] = xn
    hp = jnp.dot(xn, wn[...], preferred_element_type=jnp.float32) * dinv
    hpl[...] = hp[:, :HH]
    hpr[...] = hp[:, HH:]


def _postpre(agg2, x, b, g, beta, wn, deg):
    return pl.pallas_call(
        _postpre_body,
        grid=(NBR,),
        in_specs=_half_specs()
                 + [pl.BlockSpec((BR, H), lambda i: (i, 0))]
                 + _param_specs(3)
                 + [pl.BlockSpec((H, H), lambda i: (0, 0))]
                 + _deg_specs(),
        out_specs=[pl.BlockSpec((BR, H), lambda i: (i, 0)),
                   pl.BlockSpec((BR, HH), lambda i: (i, 0)),
                   pl.BlockSpec((BR, HH), lambda i: (i, 0))],
        out_shape=[jax.ShapeDtypeStruct((N, H), jnp.float32),
                   jax.ShapeDtypeStruct((N, HH), jnp.float32),
                   jax.ShapeDtypeStruct((N, HH), jnp.float32)],
    )(agg2, agg2, x, b, g, beta, wn, deg, deg)


def _postab_body(al, ar, x, b, g, beta, wa, wb, b1, d0, d1, xo, ao, bo):
    dinv = _dinv(d0, d1)
    agg = jnp.concatenate([al[...], ar[...]], axis=1)
    y = agg * dinv + b[...]
    xn = jnp.maximum(_ln(y, g[...], beta[...]), 0.0) + x[...]
    xo[...] = xn
    ao[...] = jnp.dot(xn, wa[...], preferred_element_type=jnp.float32) + b1[...]
    bo[...] = jnp.dot(xn, wb[...], preferred_element_type=jnp.float32)


def _postab(agg2, x, b, g, beta, wa, wb, b1, deg):
    return pl.pallas_call(
        _postab_body,
        grid=(NBR,),
        in_specs=_half_specs()
                 + [pl.BlockSpec((BR, H), lambda i: (i, 0))]
                 + _param_specs(3)
                 + [pl.BlockSpec((H, H), lambda i: (0, 0)),
                    pl.BlockSpec((H, H), lambda i: (0, 0))]
                 + _param_specs(1)
                 + _deg_specs(),
        out_specs=[pl.BlockSpec((BR, H), lambda i: (i, 0)),
                   pl.BlockSpec((BR, H), lambda i: (i, 0)),
                   pl.BlockSpec((BR, H), lambda i: (i, 0))],
        out_shape=[jax.ShapeDtypeStruct((N, H), jnp.float32),
                   jax.ShapeDtypeStruct((N, H), jnp.float32),
                   jax.ShapeDtypeStruct((N, H), jnp.float32)],
    )(agg2, agg2, x, b, g, beta, wa, wb, b1, deg, deg)


def _logits_body(c2, wx, b2, o):
    y = jnp.maximum(c2[...], 0.0)
    o[...] = jnp.dot(y, wx[...], preferred_element_type=jnp.float32) + b2[...]


def _logits(c2, wx, b2):
    return pl.pallas_call(
        _logits_body,
        grid=(E // 2 // BE2,),
        in_specs=[
            pl.BlockSpec((BE2, 2 * H), lambda i: (i, 0)),
            pl.BlockSpec((2 * H, 2), lambda i: (0, 0)),
            pl.BlockSpec((1, 1), lambda i: (0, 0)),
        ],
        out_specs=pl.BlockSpec((BE2, 2), lambda i: (i, 0)),
        out_shape=jax.ShapeDtypeStruct((E // 2, 2), jnp.float32),
    )(c2, wx, b2)


def kernel(node_features, edge_attr, enc_W, enc_b, enc_g, enc_beta, conv_W,
           conv_b, norm_g, norm_b, mlp_W1, mlp_b1, mlp_W2, mlp_b2, edge_index):
    r = lambda v: v.reshape(1, -1)
    L = conv_W.shape[0]

    deg = _deg(edge_index)
    x, hpl, hpr = _encpre(node_features, enc_W, r(enc_b), r(enc_g),
                          r(enc_beta), conv_W[0], deg)

    for i in range(L):
        agg2 = _agg(hpl, hpr, edge_index)
        if i < L - 1:
            x, hpl, hpr = _postpre(agg2, x, r(conv_b[i]), r(norm_g[i]),
                                   r(norm_b[i]), conv_W[i + 1], deg)
        else:
            x, a, b = _postab(agg2, x, r(conv_b[i]), r(norm_g[i]),
                              r(norm_b[i]), mlp_W1[:H], mlp_W1[H:],
                              r(mlp_b1), deg)

    c2 = _combine(a, b, edge_index)
    z = jnp.zeros((H, 1), jnp.float32)
    wx = jnp.concatenate(
        [jnp.concatenate([mlp_W2, z], axis=1),
         jnp.concatenate([z, mlp_W2], axis=1)], axis=0)
    logits2 = _logits(c2, wx, mlp_b2.reshape(1, 1))
    return (x, logits2.reshape(E))


# speedup vs baseline: 17.7291x; 1.0451x over previous
import jax
import jax.numpy as jnp
from jax import lax
from jax.experimental import pallas as pl
from jax.experimental.pallas import tpu as pltpu
from jax.experimental.pallas import tpu_sc as plsc

N = 50000
E = 800000
H = 64
D_IN = 128
NC = 2
NS = 16
HH = H // 2
EKB = 128
EPS = E // NS
NBLK = EPS // EKB
TAIL = EPS % EKB
UA = 6
EPW = E // (NC * NS)
NBLKW = EPW // EKB
TAILW = EPW % EKB
UC = 3
CH = 200
NCHUNKD = N // CH

_vmesh = plsc.VectorSubcoreMesh(core_axis_name="c", subcore_axis_name="s")
_sc_params = pltpu.CompilerParams(use_tc_tiling_on_sc=False)


def _chunks(sid, nchunk, fn):
    @pl.loop(0, nchunk // NS + 1)
    def _(t):
        idx = sid + t * NS

        @pl.when(idx < nchunk)
        def _():
            fn(idx)


def _shift(idx_row, off, k):
    @pl.loop(0, k, step=16)
    def _(i):
        idx_row[pl.ds(i, 16)] = idx_row[pl.ds(i, 16)] + off


def _sc_deg(ei_hbm, deg_hbm, zbuf, ones_v, idx_v, tidx, acc, semi, semsc):
    cid = lax.axis_index("c")
    sid = lax.axis_index("s")

    z16 = jnp.zeros((16,), jnp.float32)

    @pl.loop(0, CH)
    def _(r):
        zbuf[r, :] = z16

    _chunks(sid, NCHUNKD, lambda idx: pltpu.sync_copy(
        zbuf, acc.at[pl.ds(idx * CH, CH)]))

    lane = lax.iota(jnp.int32, 16)
    unit = jnp.where(lane == 0, jnp.float32(1.0), jnp.float32(0.0))

    @pl.loop(0, EKB)
    def _(r):
        ones_v[r, :] = unit

    plsc.subcore_barrier()

    ebase = cid * (E // NC) + sid * EPW

    @pl.loop(0, NBLKW // UC)
    def _(it):
        b0 = it * UC
        cps = [pltpu.async_copy(
            ei_hbm.at[1, pl.ds(ebase + (b0 + j) * EKB, EKB)],
            idx_v.at[j], semi.at[j]) for j in range(UC)]
        ss = []
        for j in range(UC):
            cps[j].wait()
            ss.append(pltpu.async_copy(ones_v, acc.at[idx_v.at[j]],
                                       semsc.at[j], add=True))
        for s in ss:
            s.wait()

    e0 = ebase + NBLKW * EKB
    pltpu.sync_copy(ei_hbm.at[1, pl.ds(e0, TAILW)], tidx)
    pltpu.sync_copy(ones_v.at[pl.ds(0, TAILW)], acc.at[tidx], add=True)

    plsc.subcore_barrier()

    _chunks(sid, NCHUNKD, lambda idx: pltpu.sync_copy(
        acc.at[pl.ds(idx * CH, CH)],
        deg_hbm.at[pl.ds(cid * N + idx * CH, CH)]))


def _deg(ei):
    k = pl.kernel(
        _sc_deg,
        out_type=jax.ShapeDtypeStruct((2 * N, 16), jnp.float32),
        mesh=_vmesh,
        compiler_params=_sc_params,
        scratch_types=[
            pltpu.VMEM((CH, 16), jnp.float32),
            pltpu.VMEM((EKB, 16), jnp.float32),
            pltpu.VMEM((UC, EKB), jnp.int32),
            pltpu.VMEM((TAILW,), jnp.int32),
            pltpu.VMEM_SHARED((N, 16), jnp.float32),
            pltpu.SemaphoreType.DMA((UC,)),
            pltpu.SemaphoreType.DMA((UC,)),
        ],
    )
    return k(ei)


def _sc_agg(hpl_hbm, hpr_hbm, ei_hbm, agg_hbm, sidx, didx, rows,
            tsidx, tdidx, trows, acc, semi, semg, semsc):
    cid = lax.axis_index("c")
    sid = lax.axis_index("s")
    rbase = cid * N

    def work(tab):
        _chunks(sid, NCHUNKD, lambda idx: pltpu.sync_copy(
            tab.at[pl.ds(idx * CH, CH)], acc.at[pl.ds(idx * CH, CH)]))

        plsc.subcore_barrier()

        ebase = sid * EPS

        @pl.loop(0, NBLK // UA)
        def _(it):
            b0 = it * UA
            cps = []
            for j in range(UA):
                e0 = ebase + (b0 + j) * EKB
                c1 = pltpu.async_copy(ei_hbm.at[0, pl.ds(e0, EKB)],
                                      sidx.at[j], semi.at[j])
                c2 = pltpu.async_copy(ei_hbm.at[1, pl.ds(e0, EKB)],
                                      didx.at[j], semi.at[j])
                cps.append((c1, c2))
            gs = []
            for j in range(UA):
                cps[j][0].wait()
                gs.append(pltpu.async_copy(tab.at[sidx.at[j]], rows.at[j],
                                           semg.at[j]))
            ss = []
            for j in range(UA):
                cps[j][1].wait()
                gs[j].wait()
                ss.append(pltpu.async_copy(rows.at[j], acc.at[didx.at[j]],
                                           semsc.at[j], add=True))
            for s in ss:
                s.wait()

        e0 = ebase + NBLK * EKB
        pltpu.sync_copy(ei_hbm.at[0, pl.ds(e0, TAIL)], tsidx)
        pltpu.sync_copy(ei_hbm.at[1, pl.ds(e0, TAIL)], tdidx)
        pltpu.sync_copy(tab.at[tsidx], trows)
        pltpu.sync_copy(trows, acc.at[tdidx], add=True)

    @pl.when(cid == 0)
    def _():
        work(hpl_hbm)

    @pl.when(cid == 1)
    def _():
        work(hpr_hbm)

    plsc.subcore_barrier()

    _chunks(sid, NCHUNKD, lambda idx: pltpu.sync_copy(
        acc.at[pl.ds(idx * CH, CH)],
        agg_hbm.at[pl.ds(rbase + idx * CH, CH)]))


def _agg(hpl, hpr, ei):
    k = pl.kernel(
        _sc_agg,
        out_type=jax.ShapeDtypeStruct((2 * N, HH), jnp.float32),
        mesh=_vmesh,
        compiler_params=_sc_params,
        scratch_types=[
            pltpu.VMEM((UA, EKB), jnp.int32),
            pltpu.VMEM((UA, EKB), jnp.int32),
            pltpu.VMEM((UA, EKB, HH), jnp.float32),
            pltpu.VMEM((TAIL,), jnp.int32),
            pltpu.VMEM((TAIL,), jnp.int32),
            pltpu.VMEM((TAIL, HH), jnp.float32),
            pltpu.VMEM_SHARED((N, HH), jnp.float32),
            pltpu.SemaphoreType.DMA((UA,)),
            pltpu.SemaphoreType.DMA((UA,)),
            pltpu.SemaphoreType.DMA((UA,)),
        ],
    )
    return k(hpl, hpr, ei)


def _sc_combine(a_hbm, b_hbm, ei_hbm, c_hbm, sidx, didx,
                arows0, arows1, arows2, brows0, brows1, brows2,
                crows0, crows1, crows2,
                tsidx, tdidx, tarows, tbrows, tcrows,
                semi, semga, semgb, semo):
    cid = lax.axis_index("c")
    sid = lax.axis_index("s")
    wid = sid * NC + cid
    ebase = wid * EPW
    arows = [arows0, arows1, arows2]
    brows = [brows0, brows1, brows2]
    crows = [crows0, crows1, crows2]

    def addpack(a_ref, b_ref, c_ref, k):
        @pl.loop(0, k // 2)
        def _(r2):
            for half in range(2):
                for jj in range(0, H, 16):
                    c_ref[r2, pl.ds(half * H + jj, 16)] = (
                        a_ref[2 * r2 + half, pl.ds(jj, 16)] +
                        b_ref[2 * r2 + half, pl.ds(jj, 16)])

    @pl.loop(0, NBLKW // UC)
    def _(it):
        b0 = it * UC
        cps = []
        for j in range(UC):
            e0 = ebase + (b0 + j) * EKB
            c1 = pltpu.async_copy(ei_hbm.at[0, pl.ds(e0, EKB)], sidx.at[j],
                                  semi.at[j])
            c2 = pltpu.async_copy(ei_hbm.at[1, pl.ds(e0, EKB)], didx.at[j],
                                  semi.at[j])
            cps.append((c1, c2))
        gs = []
        for j in range(UC):
            cps[j][0].wait()
            cps[j][1].wait()
            ga = pltpu.async_copy(a_hbm.at[sidx.at[j]], arows[j], semga.at[j])
            gb = pltpu.async_copy(b_hbm.at[didx.at[j]], brows[j], semgb.at[j])
            gs.append((ga, gb))
        outs = []
        for j in range(UC):
            gs[j][0].wait()
            gs[j][1].wait()
            addpack(arows[j], brows[j], crows[j], EKB)
            r0 = (ebase + (b0 + j) * EKB) // 2
            outs.append(pltpu.async_copy(crows[j],
                                         c_hbm.at[pl.ds(r0, EKB // 2)],
                                         semo.at[j]))
        for o in outs:
            o.wait()

    e0 = ebase + NBLKW * EKB
    pltpu.sync_copy(ei_hbm.at[0, pl.ds(e0, TAILW)], tsidx)
    pltpu.sync_copy(ei_hbm.at[1, pl.ds(e0, TAILW)], tdidx)
    pltpu.sync_copy(a_hbm.at[tsidx], tarows)
    pltpu.sync_copy(b_hbm.at[tdidx], tbrows)
    addpack(tarows, tbrows, tcrows, TAILW)
    pltpu.sync_copy(tcrows, c_hbm.at[pl.ds(e0 // 2, TAILW // 2)])


def _combine(a, b, ei):
    k = pl.kernel(
        _sc_combine,
        out_type=jax.ShapeDtypeStruct((E // 2, 2 * H), jnp.float32),
        mesh=_vmesh,
        compiler_params=_sc_params,
        scratch_types=[
            pltpu.VMEM((UC, EKB), jnp.int32),
            pltpu.VMEM((UC, EKB), jnp.int32),
            pltpu.VMEM((EKB, H), jnp.float32),
            pltpu.VMEM((EKB, H), jnp.float32),
            pltpu.VMEM((EKB, H), jnp.float32),
            pltpu.VMEM((EKB, H), jnp.float32),
            pltpu.VMEM((EKB, H), jnp.float32),
            pltpu.VMEM((EKB, H), jnp.float32),
            pltpu.VMEM((EKB // 2, 2 * H), jnp.float32),
            pltpu.VMEM((EKB // 2, 2 * H), jnp.float32),
            pltpu.VMEM((EKB // 2, 2 * H), jnp.float32),
            pltpu.VMEM((TAILW,), jnp.int32),
            pltpu.VMEM((TAILW,), jnp.int32),
            pltpu.VMEM((TAILW, H), jnp.float32),
            pltpu.VMEM((TAILW, H), jnp.float32),
            pltpu.VMEM((TAILW // 2, 2 * H), jnp.float32),
            pltpu.SemaphoreType.DMA((UC,)),
            pltpu.SemaphoreType.DMA((UC,)),
            pltpu.SemaphoreType.DMA((UC,)),
            pltpu.SemaphoreType.DMA((UC,)),
        ],
    )
    return k(a, b, ei)


BR = 2000
NBR = N // BR
BE2 = 8000


def _ln(y, g, b):
    mu = jnp.mean(y, axis=-1, keepdims=True)
    var = jnp.mean((y - mu) ** 2, axis=-1, keepdims=True)
    return (y - mu) * lax.rsqrt(var + 1e-5) * g + b


def _dinv(d0, d1):
    return lax.rsqrt(d0[...][:, :1] + d1[...][:, :1] + 1.0)


def _deg_specs():
    return [pl.BlockSpec((BR, 16), lambda i: (i, 0)),
            pl.BlockSpec((BR, 16), lambda i: (i + NBR, 0))]


def _half_specs():
    return [pl.BlockSpec((BR, HH), lambda i: (i, 0)),
            pl.BlockSpec((BR, HH), lambda i: (i + NBR, 0))]


def _param_specs(n):
    return [pl.BlockSpec((1, H), lambda i: (0, 0)) for _ in range(n)]


def _encpre_body(nf, w, b, g, beta, w0, d0, d1, xo, hpl, hpr):
    y = jnp.dot(nf[...], w[...], preferred_element_type=jnp.float32) + b[...]
    x = jnp.maximum(_ln(y, g[...], beta[...]), 0.0)
    xo[...] = x
    hp = jnp.dot(x, w0[...],
                 preferred_element_type=jnp.float32) * _dinv(d0, d1)
    hpl[...] = hp[:, :HH]
    hpr[...] = hp[:, HH:]


def _encpre(nf, w, b, g, beta, w0, deg):
    return pl.pallas_call(
        _encpre_body,
        grid=(NBR,),
        in_specs=[pl.BlockSpec((BR, D_IN), lambda i: (i, 0)),
                  pl.BlockSpec((D_IN, H), lambda i: (0, 0))]
                 + _param_specs(3)
                 + [pl.BlockSpec((H, H), lambda i: (0, 0))]
                 + _deg_specs(),
        out_specs=[pl.BlockSpec((BR, H), lambda i: (i, 0)),
                   pl.BlockSpec((BR, HH), lambda i: (i, 0)),
                   pl.BlockSpec((BR, HH), lambda i: (i, 0))],
        out_shape=[jax.ShapeDtypeStruct((N, H), jnp.float32),
                   jax.ShapeDtypeStruct((N, HH), jnp.float32),
                   jax.ShapeDtypeStruct((N, HH), jnp.float32)],
    )(nf, w, b, g, beta, w0, deg, deg)


def _postpre_body(al, ar, x, b, g, beta, wn, d0, d1, xo, hpl, hpr):
    dinv = _dinv(d0, d1)
    agg = jnp.concatenate([al[...], ar[...]], axis=1)
    y = agg * dinv + b[...]
    xn = jnp.maximum(_ln(y, g[...], beta[...]), 0.0) + x[...]
    xo[...] = xn
    hp = jnp.dot(xn, wn[...], preferred_element_type=jnp.float32) * dinv
    hpl[...] = hp[:, :HH]
    hpr[...] = hp[:, HH:]


def _postpre(agg2, x, b, g, beta, wn, deg):
    return pl.pallas_call(
        _postpre_body,
        grid=(NBR,),
        in_specs=_half_specs()
                 + [pl.BlockSpec((BR, H), lambda i: (i, 0))]
                 + _param_specs(3)
                 + [pl.BlockSpec((H, H), lambda i: (0, 0))]
                 + _deg_specs(),
        out_specs=[pl.BlockSpec((BR, H), lambda i: (i, 0)),
                   pl.BlockSpec((BR, HH), lambda i: (i, 0)),
                   pl.BlockSpec((BR, HH), lambda i: (i, 0))],
        out_shape=[jax.ShapeDtypeStruct((N, H), jnp.float32),
                   jax.ShapeDtypeStruct((N, HH), jnp.float32),
                   jax.ShapeDtypeStruct((N, HH), jnp.float32)],
    )(agg2, agg2, x, b, g, beta, wn, deg, deg)


def _postab_body(al, ar, x, b, g, beta, wa, wb, b1, d0, d1, xo, ao, bo):
    dinv = _dinv(d0, d1)
    agg = jnp.concatenate([al[...], ar[...]], axis=1)
    y = agg * dinv + b[...]
    xn = jnp.maximum(_ln(y, g[...], beta[...]), 0.0) + x[...]
    xo[...] = xn
    ao[...] = jnp.dot(xn, wa[...], preferred_element_type=jnp.float32) + b1[...]
    bo[...] = jnp.dot(xn, wb[...], preferred_element_type=jnp.float32)


def _postab(agg2, x, b, g, beta, wa, wb, b1, deg):
    return pl.pallas_call(
        _postab_body,
        grid=(NBR,),
        in_specs=_half_specs()
                 + [pl.BlockSpec((BR, H), lambda i: (i, 0))]
                 + _param_specs(3)
                 + [pl.BlockSpec((H, H), lambda i: (0, 0)),
                    pl.BlockSpec((H, H), lambda i: (0, 0))]
                 + _param_specs(1)
                 + _deg_specs(),
        out_specs=[pl.BlockSpec((BR, H), lambda i: (i, 0)),
                   pl.BlockSpec((BR, H), lambda i: (i, 0)),
                   pl.BlockSpec((BR, H), lambda i: (i, 0))],
        out_shape=[jax.ShapeDtypeStruct((N, H), jnp.float32),
                   jax.ShapeDtypeStruct((N, H), jnp.float32),
                   jax.ShapeDtypeStruct((N, H), jnp.float32)],
    )(agg2, agg2, x, b, g, beta, wa, wb, b1, deg, deg)


def _logits_body(c2, wx, b2, o):
    y = jnp.maximum(c2[...], 0.0)
    o[...] = jnp.dot(y, wx[...], preferred_element_type=jnp.float32) + b2[...]


def _logits(c2, wx, b2):
    return pl.pallas_call(
        _logits_body,
        grid=(E // 2 // BE2,),
        in_specs=[
            pl.BlockSpec((BE2, 2 * H), lambda i: (i, 0)),
            pl.BlockSpec((2 * H, 2), lambda i: (0, 0)),
            pl.BlockSpec((1, 1), lambda i: (0, 0)),
        ],
        out_specs=pl.BlockSpec((BE2, 2), lambda i: (i, 0)),
        out_shape=jax.ShapeDtypeStruct((E // 2, 2), jnp.float32),
    )(c2, wx, b2)


def kernel(node_features, edge_attr, enc_W, enc_b, enc_g, enc_beta, conv_W,
           conv_b, norm_g, norm_b, mlp_W1, mlp_b1, mlp_W2, mlp_b2, edge_index):
    r = lambda v: v.reshape(1, -1)
    L = conv_W.shape[0]

    deg = _deg(edge_index)
    x, hpl, hpr = _encpre(node_features, enc_W, r(enc_b), r(enc_g),
                          r(enc_beta), conv_W[0], deg)

    for i in range(L):
        agg2 = _agg(hpl, hpr, edge_index)
        if i < L - 1:
            x, hpl, hpr = _postpre(agg2, x, r(conv_b[i]), r(norm_g[i]),
                                   r(norm_b[i]), conv_W[i + 1], deg)
        else:
            x, a, b = _postab(agg2, x, r(conv_b[i]), r(norm_g[i]),
                              r(norm_b[i]), mlp_W1[:H], mlp_W1[H:],
                              r(mlp_b1), deg)

    c2 = _combine(a, b, edge_index)
    z = jnp.zeros((H, 1), jnp.float32)
    wx = jnp.concatenate(
        [jnp.concatenate([mlp_W2, z], axis=1),
         jnp.concatenate([z, mlp_W2], axis=1)], axis=0)
    logits2 = _logits(c2, wx, mlp_b2.reshape(1, 1))
    return (x, logits2.reshape(E))
